# Initial kernel scaffold; baseline (speedup 1.0000x reference)
import jax
import jax.numpy as jnp
from jax.experimental import pallas as pl


def kernel(inputs, edge_index, W_in, b_in, spatial_emb, temporal_emb, Wg, a_src, a_dst, W_agg, b_agg, W_out1, b_out1, W_out2, b_out2):
    raise NotImplementedError("write your pallas kernel here")


# speedup vs baseline: 1.0544x; 1.0544x over previous
import functools

import jax
import jax.numpy as jnp
from jax.experimental import pallas as pl
from jax.experimental.pallas import tpu as pltpu

B, T, N, DIN = 8, 13, 1200, 2
D, H, L, S = 128, 4, 2, 4
DH = D // H
SEQ_OUT = 12
M = B * S * N
E = 614400
CKPNT = [4, 7, 10, 13]


def _embed_body(in_ref, w_ref, b_ref, sp_ref, te_ref, o_ref):
    t = pl.program_id(0) % T
    x2 = in_ref[0]
    w = w_ref[...]
    onehot = (jax.lax.broadcasted_iota(jnp.int32, (T, 1), 0) == t).astype(jnp.float32)
    te_col = jnp.dot(te_ref[...], onehot,
                     preferred_element_type=jnp.float32)
    x = (x2[:, 0:1] * w[0:1, :] + x2[:, 1:2] * w[1:1 + 1, :]
         + b_ref[...][None, :] + sp_ref[...] + te_col)
    o_ref[0] = x


def _embed(inputs, W_in, b_in, spatial_emb, temporal_emb):
    flat_in = inputs.reshape(B * T, N, DIN)
    grid = (B * T,)
    out = pl.pallas_call(
        _embed_body,
        grid=grid,
        in_specs=[
            pl.BlockSpec((1, N, DIN), lambda i: (i, 0, 0)),
            pl.BlockSpec((DIN, D), lambda i: (0, 0)),
            pl.BlockSpec((D,), lambda i: (0,)),
            pl.BlockSpec((N, D), lambda i: (0, 0)),
            pl.BlockSpec((N, T), lambda i: (0, 0)),
        ],
        out_specs=pl.BlockSpec((1, N, D), lambda i: (i, 0, 0)),
        out_shape=jax.ShapeDtypeStruct((B * T, N, D), jnp.float32),
    )(flat_in, W_in, b_in, spatial_emb, temporal_emb)
    return out.reshape(B, T, N, D)


def _gat(h, src, dst, Wg, a_src, a_dst):
    for l in range(L):
        hp = (h @ Wg[l]).reshape(M, H, DH)
        es = jnp.sum(hp * a_src[l], axis=-1)
        ed = jnp.sum(hp * a_dst[l], axis=-1)
        e = jax.nn.leaky_relu(es[src] + ed[dst], 0.2)
        ex = jnp.exp(e)
        den = jax.ops.segment_sum(ex, dst, num_segments=M)
        num = jax.ops.segment_sum(hp[src] * ex[:, :, None], dst, num_segments=M)
        out = (num / (den[:, :, None] + 1e-16)).reshape(M, D)
        h = jax.nn.elu(out) if l < L - 1 else out
    return h


def kernel(inputs, edge_index, W_in, b_in, spatial_emb, temporal_emb, Wg, a_src,
           a_dst, W_agg, b_agg, W_out1, b_out1, W_out2, b_out2):
    src, dst = edge_index[0], edge_index[1]
    x = _embed(inputs, W_in, b_in, spatial_emb, temporal_emb)
    last = None
    left = 0
    for i, right in enumerate(CKPNT):
        sl = x[:, left:right]
        if i != 0:
            sl = jnp.concatenate([last, sl], axis=1)
        residual = sl
        ret = _gat(sl.reshape(M, D), src, dst, Wg, a_src, a_dst).reshape(B, S, N, D)
        last = ret + residual
        w = jax.nn.softmax(jnp.tanh(last @ W_agg + b_agg), axis=1)
        last = jnp.sum(w * last, axis=1, keepdims=True)
        mu = jnp.mean(last, axis=(1, 2, 3), keepdims=True)
        var = jnp.var(last, axis=(1, 2, 3), keepdims=True)
        last = (last - mu) / jnp.sqrt(var + 1e-5)
        left = right
    t = jnp.swapaxes(last, 1, 3)
    out = jax.nn.relu(t @ W_out1 + b_out1)
    out = jnp.swapaxes(out, 1, 3)
    out = out @ W_out2 + b_out2
    return out


# speedup vs baseline: 15.7764x; 14.9629x over previous
import functools

import jax
import jax.numpy as jnp
from jax import lax
from jax.experimental import pallas as pl
from jax.experimental.pallas import tpu as pltpu
from jax.experimental.pallas import tpu_sc as plsc

B, T, N, DIN = 8, 13, 1200, 2
D, H, L, S = 128, 4, 2, 4
DH = D // H
SEQ_OUT = 12
M = B * S * N
E = 614400
CKPNT = [4, 7, 10, 13]

NS = 8
SW = 16
CHUNK = 1280
BK = 128
EPT_S = E // 16
EPT_D = E // 32
ROWS_PT = M // 16
WBC = 600

_sc_mesh = plsc.VectorSubcoreMesh(core_axis_name="c", subcore_axis_name="s")


def _embed_body(in_ref, w_ref, b_ref, sp_ref, te_ref, o_ref):
    t = pl.program_id(0) % T
    x2 = in_ref[0]
    w = w_ref[...]
    onehot = (lax.broadcasted_iota(jnp.int32, (T, 1), 0) == t).astype(jnp.float32)
    te_col = jnp.dot(te_ref[...], onehot,
                     preferred_element_type=jnp.float32)
    x = (x2[:, 0:1] * w[0:1, :] + x2[:, 1:2] * w[1:2, :]
         + b_ref[...][None, :] + sp_ref[...] + te_col)
    o_ref[0] = x


def _embed(inputs, W_in, b_in, spatial_emb, temporal_emb):
    flat_in = inputs.reshape(B * T, N, DIN)
    out = pl.pallas_call(
        _embed_body,
        grid=(B * T,),
        in_specs=[
            pl.BlockSpec((1, N, DIN), lambda i: (i, 0, 0)),
            pl.BlockSpec((DIN, D), lambda i: (0, 0)),
            pl.BlockSpec((D,), lambda i: (0,)),
            pl.BlockSpec((N, D), lambda i: (0, 0)),
            pl.BlockSpec((N, T), lambda i: (0, 0)),
        ],
        out_specs=pl.BlockSpec((1, N, D), lambda i: (i, 0, 0)),
        out_shape=jax.ShapeDtypeStruct((B * T, N, D), jnp.float32),
    )(flat_in, W_in, b_in, spatial_emb, temporal_emb)
    return out.reshape(B, T, N, D)


def _den_expand(den16):
    hsel = lax.broadcasted_iota(jnp.int32, (16, D), 0)
    csel = lax.broadcasted_iota(jnp.int32, (16, D), 1)
    q = (hsel == csel // DH).astype(jnp.float32)
    return jnp.dot(den16, q, preferred_element_type=jnp.float32)


_PROJ_R = 768


def _proj_body(finish, h_ref, den_ref, wg_ref, av_ref,
               hpq_ref, edq_ref, es_ref, ed_ref):
    h = h_ref[...]
    if finish:
        h = h / (_den_expand(den_ref[0] + den_ref[1]) + 1e-16)
        h = jnp.where(h > 0, h, jnp.exp(jnp.minimum(h, 0.0)) - 1.0)
    hp = jnp.dot(h, wg_ref[...], preferred_element_type=jnp.float32)
    row = lax.broadcasted_iota(jnp.int32, (D, 16), 0)
    col = lax.broadcasted_iota(jnp.int32, (D, 16), 1)
    seg = ((row // DH == col) & (col < H)).astype(jnp.float32)
    av = av_ref[...]
    es = jnp.dot(hp * av[0:1, :], seg, preferred_element_type=jnp.float32)
    ed = jnp.dot(hp * av[1:2, :], seg, preferred_element_type=jnp.float32)
    es_ref[...] = es
    ed_ref[...] = ed
    r16 = lax.broadcasted_iota(jnp.int32, (16, 16), 0)
    for s in range(NS):
        sel = (r16 == s // 2).astype(jnp.float32)
        es_rep = jnp.dot(es, sel, preferred_element_type=jnp.float32)
        ed_rep = jnp.dot(ed, sel, preferred_element_type=jnp.float32)
        hpq_ref[s] = jnp.concatenate([hp[:, s * SW:(s + 1) * SW], es_rep], axis=1)
        edq_ref[s] = ed_rep


def _proj_tables(h, den2, Wg_l, a_src_l, a_dst_l, finish):
    av = jnp.stack([a_src_l.reshape(D), a_dst_l.reshape(D)])
    return pl.pallas_call(
        functools.partial(_proj_body, finish),
        grid=(M // _PROJ_R,),
        in_specs=[
            pl.BlockSpec((_PROJ_R, D), lambda i: (i, 0)),
            pl.BlockSpec((2, _PROJ_R, 16), lambda i: (0, i, 0)),
            pl.BlockSpec((D, D), lambda i: (0, 0)),
            pl.BlockSpec((2, D), lambda i: (0, 0)),
        ],
        out_specs=[
            pl.BlockSpec((NS, _PROJ_R, 2 * SW), lambda i: (0, i, 0)),
            pl.BlockSpec((NS, _PROJ_R, SW), lambda i: (0, i, 0)),
            pl.BlockSpec((_PROJ_R, 16), lambda i: (i, 0)),
            pl.BlockSpec((_PROJ_R, 16), lambda i: (i, 0)),
        ],
        out_shape=[
            jax.ShapeDtypeStruct((NS, M, 2 * SW), jnp.float32),
            jax.ShapeDtypeStruct((NS, M, SW), jnp.float32),
            jax.ShapeDtypeStruct((M, 16), jnp.float32),
            jax.ShapeDtypeStruct((M, 16), jnp.float32),
        ],
    )(h, den2, Wg_l, av)


def _combine_body(ret_ref, den_ref, res_ref, wagg_ref, bagg_ref, o_ref):
    ret = ret_ref[0] / (_den_expand(den_ref[0, 0] + den_ref[1, 0]) + 1e-16)
    x = ret.reshape(S, N, D) + res_ref[0]
    wv = wagg_ref[...]
    score = jnp.tanh(jnp.sum(x * wv[None, :, :], axis=-1) + bagg_ref[0])
    mx = jnp.max(score, axis=0, keepdims=True)
    ex = jnp.exp(score - mx)
    w = ex / jnp.sum(ex, axis=0, keepdims=True)
    agg = jnp.sum(x * w[:, :, None], axis=0)
    mu = jnp.mean(agg)
    var = jnp.mean((agg - mu) ** 2)
    o_ref[0] = (agg - mu) * lax.rsqrt(var + 1e-5)


def _combine(ret, den2, res, W_agg, b_agg):
    return pl.pallas_call(
        _combine_body,
        grid=(B,),
        in_specs=[
            pl.BlockSpec((1, S * N, D), lambda i: (i, 0, 0)),
            pl.BlockSpec((2, 1, S * N, 16), lambda i: (0, i, 0, 0)),
            pl.BlockSpec((1, S, N, D), lambda i: (i, 0, 0, 0)),
            pl.BlockSpec((1, D), lambda i: (0, 0)),
            pl.BlockSpec(memory_space=pltpu.SMEM),
        ],
        out_specs=pl.BlockSpec((1, N, D), lambda i: (i, 0, 0)),
        out_shape=jax.ShapeDtypeStruct((B, N, D), jnp.float32),
    )(ret.reshape(B, S * N, D), den2.reshape(2, B, S * N, 16), res,
      W_agg.reshape(1, D), b_agg)


def _decode_body(last_ref, w1_ref, b1_ref, w2_ref, b2_ref, o_ref):
    x = last_ref[0]
    w1 = w1_ref[...]
    b1 = b1_ref[...]
    w2 = w2_ref[...]
    for j in range(SEQ_OUT):
        rj = jnp.maximum(x * w1[0, j] + b1[0, j], 0.0)
        o_ref[0, j, :] = jnp.sum(rj * w2, axis=-1) + b2_ref[0]


def _decode(last, W_out1, b_out1, W_out2, b_out2):
    return pl.pallas_call(
        _decode_body,
        grid=(B,),
        in_specs=[
            pl.BlockSpec((1, N, D), lambda i: (i, 0, 0)),
            pl.BlockSpec((1, SEQ_OUT), lambda i: (0, 0)),
            pl.BlockSpec((1, SEQ_OUT), lambda i: (0, 0)),
            pl.BlockSpec((1, D), lambda i: (0, 0)),
            pl.BlockSpec(memory_space=pltpu.SMEM),
        ],
        out_specs=pl.BlockSpec((1, SEQ_OUT, N), lambda i: (i, 0, 0)),
        out_shape=jax.ShapeDtypeStruct((B, SEQ_OUT, N), jnp.float32),
    )(last, W_out1, b_out1.reshape(1, SEQ_OUT), W_out2.reshape(1, D), b_out2)


def _edge_body(hpq_hbm, edq_hbm, es_hbm, ed_hbm, src_hbm, dst_hbm,
               out_hbm, den_hbm,
               srcc_v, idxe_v, dl_v, a_v, e_v, sc_v, wb_v, acc_sh, semg, seme):
    cid = lax.axis_index("c")
    sid = lax.axis_index("s")

    def zero_wb():
        z = jnp.zeros((16,), jnp.float32)
        def zr(r, _):
            wb_v[r, pl.ds(0, 16)] = z
            return 0
        lax.fori_loop(0, WBC, zr, 0)

    def zero_acc():
        for i in range(ROWS_PT // WBC):
            pltpu.sync_copy(wb_v,
                            acc_sh.at[pl.ds(sid * ROWS_PT + i * WBC, WBC)])

    def stage_chunk(e0, tab_off, den_mode):
        pltpu.sync_copy(src_hbm.at[pl.ds(e0, CHUNK)], srcc_v)
        pltpu.sync_copy(dst_hbm.at[pl.ds(e0, CHUNK)], idxe_v)

        def fix(j, _):
            sv = srcc_v[pl.ds(j * 16, 16)]
            dv = idxe_v[pl.ds(j * 16, 16)]
            dl_v[j // 8, pl.ds((j % 8) * 16, 16)] = dv
            if not den_mode:
                srcc_v[pl.ds(j * 16, 16)] = sv + tab_off
                idxe_v[pl.ds(j * 16, 16)] = dv + tab_off
            return 0
        lax.fori_loop(0, CHUNK // 16, fix, 0)

    def run_pass(tab_off, n_chunks, base, den_mode):
        def chunk(ci, _):
            stage_chunk(base + ci * CHUNK, tab_off, den_mode)

            def batch(k, _):
                if den_mode:
                    pltpu.async_copy(es_hbm.at[srcc_v.at[pl.ds(k * BK, BK)]],
                                     e_v, semg).wait()
                    pltpu.async_copy(ed_hbm.at[idxe_v.at[pl.ds(k * BK, BK)]],
                                     sc_v, seme).wait()
                else:
                    pltpu.async_copy(hpq_hbm.at[srcc_v.at[pl.ds(k * BK, BK)]],
                                     a_v, semg).wait()
                    pltpu.async_copy(edq_hbm.at[idxe_v.at[pl.ds(k * BK, BK)]],
                                     e_v, seme).wait()

                def edge(e, _):
                    if den_mode:
                        t = e_v[e, pl.ds(0, 16)] + sc_v[e, pl.ds(0, 16)]
                        ex = jnp.exp(jnp.where(t >= 0.0, t, 0.2 * t))
                        sc_v[e, pl.ds(0, 16)] = ex
                    else:
                        t = a_v[e, pl.ds(SW, 16)] + e_v[e, pl.ds(0, 16)]
                        ex = jnp.exp(jnp.where(t >= 0.0, t, 0.2 * t))
                        sc_v[e, pl.ds(0, 16)] = a_v[e, pl.ds(0, 16)] * ex
                    return 0
                lax.fori_loop(0, BK, edge, 0)

                pltpu.sync_copy(sc_v, acc_sh.at[dl_v.at[k]], add=True)
                return 0
            lax.fori_loop(0, CHUNK // BK, batch, 0)
            return 0
        lax.fori_loop(0, n_chunks, chunk, 0)

    def writeback(dst_ref):
        r0 = sid * ROWS_PT
        def wchunk(ci, _):
            pltpu.sync_copy(acc_sh.at[pl.ds(r0 + ci * WBC, WBC)], wb_v)
            pltpu.sync_copy(wb_v, dst_ref.at[pl.ds(r0 + ci * WBC, WBC)])
            return 0
        lax.fori_loop(0, ROWS_PT // WBC, wchunk, 0)

    zero_wb()
    for q in range(NS // 2):
        zero_acc()
        plsc.subcore_barrier()
        run_pass((cid * (NS // 2) + q) * M, EPT_S // CHUNK, sid * EPT_S,
                 den_mode=False)
        plsc.subcore_barrier()
        writeback(out_hbm.at[cid * (NS // 2) + q])
        plsc.subcore_barrier()
        zero_wb()
    zero_acc()
    plsc.subcore_barrier()
    run_pass(0, EPT_D // CHUNK, cid * (E // 2) + sid * EPT_D, den_mode=True)
    plsc.subcore_barrier()
    writeback(den_hbm.at[cid])


def _edge_pass(hpq, edq, es16, ed16, src, dst):
    f = pl.kernel(
        _edge_body,
        out_type=[
            jax.ShapeDtypeStruct((NS, M, SW), jnp.float32),
            jax.ShapeDtypeStruct((2, M, 16), jnp.float32),
        ],
        mesh=_sc_mesh,
        scratch_types=[
            pltpu.VMEM((CHUNK,), jnp.int32),
            pltpu.VMEM((CHUNK,), jnp.int32),
            pltpu.VMEM((CHUNK // BK, BK), jnp.int32),
            pltpu.VMEM((BK, 2 * SW), jnp.float32),
            pltpu.VMEM((BK, 16), jnp.float32),
            pltpu.VMEM((BK, 16), jnp.float32),
            pltpu.VMEM((WBC, 16), jnp.float32),
            pltpu.VMEM_SHARED((M, 16), jnp.float32),
            pltpu.SemaphoreType.DMA,
            pltpu.SemaphoreType.DMA,
        ],
        compiler_params=pltpu.CompilerParams(use_tc_tiling_on_sc=False),
    )
    return f(hpq.reshape(NS * M, 2 * SW), edq.reshape(NS * M, SW),
             es16, ed16, src, dst)


def kernel(inputs, edge_index, W_in, b_in, spatial_emb, temporal_emb, Wg, a_src,
           a_dst, W_agg, b_agg, W_out1, b_out1, W_out2, b_out2):
    src, dst = edge_index[0], edge_index[1]
    x = _embed(inputs, W_in, b_in, spatial_emb, temporal_emb)

    last = None
    left = 0
    for i, right in enumerate(CKPNT):
        if i == 0:
            h0 = x[:, left:right]
        else:
            h0 = jnp.concatenate([last[:, None], x[:, left:right]], axis=1)
        res = h0.reshape(M, D)

        cur = res
        den2 = jnp.zeros((2, M, 16), jnp.float32)
        for l in range(L):
            hpq, edq, es16, ed16 = _proj_tables(
                cur, den2, Wg[l], a_src[l], a_dst[l], finish=(l > 0))
            out8, den2 = _edge_pass(hpq, edq, es16, ed16, src, dst)
            cur = out8.transpose(1, 0, 2).reshape(M, D)
        last = _combine(cur.reshape(B, S * N, D), den2, h0, W_agg, b_agg)
        left = right

    out = _decode(last, W_out1, b_out1, W_out2, b_out2)
    return out.reshape(B, SEQ_OUT, N, 1)


# speedup vs baseline: 25.3188x; 1.6049x over previous
import functools

import jax
import jax.numpy as jnp
from jax import lax
from jax.experimental import pallas as pl
from jax.experimental.pallas import tpu as pltpu
from jax.experimental.pallas import tpu_sc as plsc

B, T, N, DIN = 8, 13, 1200, 2
D, H, L, S = 128, 4, 2, 4
DH = D // H
SEQ_OUT = 12
M = B * S * N
E = 614400
CKPNT = [4, 7, 10, 13]

NS = 8
SW = 16
CHUNK = 1280
BK = 128
EPT_S = E // 16
EPT_D = E // 32
ROWS_PT = M // 16
WBC = 600

_sc_mesh = plsc.VectorSubcoreMesh(core_axis_name="c", subcore_axis_name="s")


def _embed_body(in_ref, w_ref, b_ref, sp_ref, te_ref, o_ref):
    t = pl.program_id(0) % T
    x2 = in_ref[0]
    w = w_ref[...]
    onehot = (lax.broadcasted_iota(jnp.int32, (T, 1), 0) == t).astype(jnp.float32)
    te_col = jnp.dot(te_ref[...], onehot,
                     preferred_element_type=jnp.float32)
    x = (x2[:, 0:1] * w[0:1, :] + x2[:, 1:2] * w[1:2, :]
         + b_ref[...][None, :] + sp_ref[...] + te_col)
    o_ref[0] = x


def _embed(inputs, W_in, b_in, spatial_emb, temporal_emb):
    flat_in = inputs.reshape(B * T, N, DIN)
    out = pl.pallas_call(
        _embed_body,
        grid=(B * T,),
        in_specs=[
            pl.BlockSpec((1, N, DIN), lambda i: (i, 0, 0)),
            pl.BlockSpec((DIN, D), lambda i: (0, 0)),
            pl.BlockSpec((D,), lambda i: (0,)),
            pl.BlockSpec((N, D), lambda i: (0, 0)),
            pl.BlockSpec((N, T), lambda i: (0, 0)),
        ],
        out_specs=pl.BlockSpec((1, N, D), lambda i: (i, 0, 0)),
        out_shape=jax.ShapeDtypeStruct((B * T, N, D), jnp.float32),
    )(flat_in, W_in, b_in, spatial_emb, temporal_emb)
    return out.reshape(B, T, N, D)


def _den_expand(den16):
    hsel = lax.broadcasted_iota(jnp.int32, (16, D), 0)
    csel = lax.broadcasted_iota(jnp.int32, (16, D), 1)
    q = (hsel == csel // DH).astype(jnp.float32)
    return jnp.dot(den16, q, preferred_element_type=jnp.float32)


_PROJ_R = 768


def _proj_body(finish, h_ref, den_ref, wg_ref, av_ref,
               hpq_ref, edq_ref, es_ref, ed_ref):
    h = h_ref[...]
    if finish:
        h = h / (_den_expand(den_ref[0] + den_ref[1]) + 1e-16)
        h = jnp.where(h > 0, h, jnp.exp(jnp.minimum(h, 0.0)) - 1.0)
    hp = jnp.dot(h, wg_ref[...], preferred_element_type=jnp.float32)
    row = lax.broadcasted_iota(jnp.int32, (D, 16), 0)
    col = lax.broadcasted_iota(jnp.int32, (D, 16), 1)
    seg = ((row // DH == col) & (col < H)).astype(jnp.float32)
    av = av_ref[...]
    es = jnp.dot(hp * av[0:1, :], seg, preferred_element_type=jnp.float32)
    ed = jnp.dot(hp * av[1:2, :], seg, preferred_element_type=jnp.float32)
    es_ref[...] = es
    ed_ref[...] = ed
    r16 = lax.broadcasted_iota(jnp.int32, (16, 16), 0)
    for s in range(NS):
        sel = (r16 == s // 2).astype(jnp.float32)
        es_rep = jnp.dot(es, sel, preferred_element_type=jnp.float32)
        ed_rep = jnp.dot(ed, sel, preferred_element_type=jnp.float32)
        hpq_ref[s] = jnp.concatenate([hp[:, s * SW:(s + 1) * SW], es_rep], axis=1)
        edq_ref[s] = ed_rep


def _proj_tables(h, den2, Wg_l, a_src_l, a_dst_l, finish):
    av = jnp.stack([a_src_l.reshape(D), a_dst_l.reshape(D)])
    return pl.pallas_call(
        functools.partial(_proj_body, finish),
        grid=(M // _PROJ_R,),
        in_specs=[
            pl.BlockSpec((_PROJ_R, D), lambda i: (i, 0)),
            pl.BlockSpec((2, _PROJ_R, 16), lambda i: (0, i, 0)),
            pl.BlockSpec((D, D), lambda i: (0, 0)),
            pl.BlockSpec((2, D), lambda i: (0, 0)),
        ],
        out_specs=[
            pl.BlockSpec((NS, _PROJ_R, 2 * SW), lambda i: (0, i, 0)),
            pl.BlockSpec((NS, _PROJ_R, SW), lambda i: (0, i, 0)),
            pl.BlockSpec((_PROJ_R, 16), lambda i: (i, 0)),
            pl.BlockSpec((_PROJ_R, 16), lambda i: (i, 0)),
        ],
        out_shape=[
            jax.ShapeDtypeStruct((NS, M, 2 * SW), jnp.float32),
            jax.ShapeDtypeStruct((NS, M, SW), jnp.float32),
            jax.ShapeDtypeStruct((M, 16), jnp.float32),
            jax.ShapeDtypeStruct((M, 16), jnp.float32),
        ],
    )(h, den2, Wg_l, av)


def _combine_body(ret_ref, den_ref, res_ref, wagg_ref, bagg_ref, o_ref):
    ret = ret_ref[0] / (_den_expand(den_ref[0, 0] + den_ref[1, 0]) + 1e-16)
    x = ret.reshape(S, N, D) + res_ref[0]
    wv = wagg_ref[...]
    score = jnp.tanh(jnp.sum(x * wv[None, :, :], axis=-1) + bagg_ref[0])
    mx = jnp.max(score, axis=0, keepdims=True)
    ex = jnp.exp(score - mx)
    w = ex / jnp.sum(ex, axis=0, keepdims=True)
    agg = jnp.sum(x * w[:, :, None], axis=0)
    mu = jnp.mean(agg)
    var = jnp.mean((agg - mu) ** 2)
    o_ref[0] = (agg - mu) * lax.rsqrt(var + 1e-5)


def _combine(ret, den2, res, W_agg, b_agg):
    return pl.pallas_call(
        _combine_body,
        grid=(B,),
        in_specs=[
            pl.BlockSpec((1, S * N, D), lambda i: (i, 0, 0)),
            pl.BlockSpec((2, 1, S * N, 16), lambda i: (0, i, 0, 0)),
            pl.BlockSpec((1, S, N, D), lambda i: (i, 0, 0, 0)),
            pl.BlockSpec((1, D), lambda i: (0, 0)),
            pl.BlockSpec(memory_space=pltpu.SMEM),
        ],
        out_specs=pl.BlockSpec((1, N, D), lambda i: (i, 0, 0)),
        out_shape=jax.ShapeDtypeStruct((B, N, D), jnp.float32),
    )(ret.reshape(B, S * N, D), den2.reshape(2, B, S * N, 16), res,
      W_agg.reshape(1, D), b_agg)


def _decode_body(last_ref, w1_ref, b1_ref, w2_ref, b2_ref, o_ref):
    x = last_ref[0]
    w1 = w1_ref[...]
    b1 = b1_ref[...]
    w2 = w2_ref[...]
    for j in range(SEQ_OUT):
        rj = jnp.maximum(x * w1[0, j] + b1[0, j], 0.0)
        o_ref[0, j, :] = jnp.sum(rj * w2, axis=-1) + b2_ref[0]


def _decode(last, W_out1, b_out1, W_out2, b_out2):
    return pl.pallas_call(
        _decode_body,
        grid=(B,),
        in_specs=[
            pl.BlockSpec((1, N, D), lambda i: (i, 0, 0)),
            pl.BlockSpec((1, SEQ_OUT), lambda i: (0, 0)),
            pl.BlockSpec((1, SEQ_OUT), lambda i: (0, 0)),
            pl.BlockSpec((1, D), lambda i: (0, 0)),
            pl.BlockSpec(memory_space=pltpu.SMEM),
        ],
        out_specs=pl.BlockSpec((1, SEQ_OUT, N), lambda i: (i, 0, 0)),
        out_shape=jax.ShapeDtypeStruct((B, SEQ_OUT, N), jnp.float32),
    )(last, W_out1, b_out1.reshape(1, SEQ_OUT), W_out2.reshape(1, D), b_out2)


def _edge_body(hpq_hbm, edq_hbm, es_hbm, ed_hbm, src_hbm, dst_hbm,
               out_hbm, den_hbm,
               srcc_v, idxe_v, dl_v, a_v, e_v, es2_v, sc_v, wb_v, acc_sh,
               sga0, sga1, sge0, sge1):
    cid = lax.axis_index("c")
    sid = lax.axis_index("s")

    def zero_wb():
        z = jnp.zeros((16,), jnp.float32)
        def zr(r, _):
            wb_v[r, pl.ds(0, 16)] = z
            return 0
        lax.fori_loop(0, WBC, zr, 0)

    def zero_acc():
        for i in range(ROWS_PT // WBC):
            pltpu.sync_copy(wb_v,
                            acc_sh.at[pl.ds(sid * ROWS_PT + i * WBC, WBC)])

    def stage_chunk(e0, tab_off, den_mode):
        pltpu.sync_copy(src_hbm.at[pl.ds(e0, CHUNK)], srcc_v)
        pltpu.sync_copy(dst_hbm.at[pl.ds(e0, CHUNK)], idxe_v)

        def fix(j, _):
            sv = srcc_v[pl.ds(j * 16, 16)]
            dv = idxe_v[pl.ds(j * 16, 16)]
            dl_v[j // 8, pl.ds((j % 8) * 16, 16)] = dv
            if not den_mode:
                srcc_v[pl.ds(j * 16, 16)] = sv + tab_off
                idxe_v[pl.ds(j * 16, 16)] = dv + tab_off
            return 0
        lax.fori_loop(0, CHUNK // 16, fix, 0)

    def run_pass(tab_off, n_chunks, base, den_mode):
        sga = [sga0, sga1]
        sge = [sge0, sge1]
        nb = CHUNK // BK

        def chunk(ci, _):
            stage_chunk(base + ci * CHUNK, tab_off, den_mode)
            hA = [None, None]
            hE = [None, None]

            def start(k):
                buf = k % 2
                sl = pl.ds(k * BK, BK)
                if den_mode:
                    hA[buf] = pltpu.async_copy(es_hbm.at[srcc_v.at[sl]],
                                               es2_v.at[buf], sga[buf])
                    hE[buf] = pltpu.async_copy(ed_hbm.at[idxe_v.at[sl]],
                                               e_v.at[buf], sge[buf])
                else:
                    hA[buf] = pltpu.async_copy(hpq_hbm.at[srcc_v.at[sl]],
                                               a_v.at[buf], sga[buf])
                    hE[buf] = pltpu.async_copy(edq_hbm.at[idxe_v.at[sl]],
                                               e_v.at[buf], sge[buf])

            start(0)
            for k in range(nb):
                buf = k % 2
                if k + 1 < nb:
                    start(k + 1)
                hA[buf].wait()
                hE[buf].wait()

                def edge(e, _, buf=buf, den_mode=den_mode):
                    if den_mode:
                        t = es2_v[buf, e, pl.ds(0, 16)] + e_v[buf, e, pl.ds(0, 16)]
                        ex = jnp.exp(jnp.where(t >= 0.0, t, 0.2 * t))
                        sc_v[e, pl.ds(0, 16)] = ex
                    else:
                        t = a_v[buf, e, pl.ds(SW, 16)] + e_v[buf, e, pl.ds(0, 16)]
                        ex = jnp.exp(jnp.where(t >= 0.0, t, 0.2 * t))
                        sc_v[e, pl.ds(0, 16)] = a_v[buf, e, pl.ds(0, 16)] * ex
                    return 0
                lax.fori_loop(0, BK, edge, 0)

                pltpu.sync_copy(sc_v, acc_sh.at[dl_v.at[k]], add=True)
            return 0
        lax.fori_loop(0, n_chunks, chunk, 0)

    def writeback(dst_ref):
        r0 = sid * ROWS_PT
        def wchunk(ci, _):
            pltpu.sync_copy(acc_sh.at[pl.ds(r0 + ci * WBC, WBC)], wb_v)
            pltpu.sync_copy(wb_v, dst_ref.at[pl.ds(r0 + ci * WBC, WBC)])
            return 0
        lax.fori_loop(0, ROWS_PT // WBC, wchunk, 0)

    zero_wb()
    for q in range(NS // 2):
        zero_acc()
        plsc.subcore_barrier()
        run_pass((cid * (NS // 2) + q) * M, EPT_S // CHUNK, sid * EPT_S,
                 den_mode=False)
        plsc.subcore_barrier()
        writeback(out_hbm.at[cid * (NS // 2) + q])
        plsc.subcore_barrier()
        zero_wb()
    zero_acc()
    plsc.subcore_barrier()
    run_pass(0, EPT_D // CHUNK, cid * (E // 2) + sid * EPT_D, den_mode=True)
    plsc.subcore_barrier()
    writeback(den_hbm.at[cid])


def _edge_pass(hpq, edq, es16, ed16, src, dst):
    f = pl.kernel(
        _edge_body,
        out_type=[
            jax.ShapeDtypeStruct((NS, M, SW), jnp.float32),
            jax.ShapeDtypeStruct((2, M, 16), jnp.float32),
        ],
        mesh=_sc_mesh,
        scratch_types=[
            pltpu.VMEM((CHUNK,), jnp.int32),
            pltpu.VMEM((CHUNK,), jnp.int32),
            pltpu.VMEM((CHUNK // BK, BK), jnp.int32),
            pltpu.VMEM((2, BK, 2 * SW), jnp.float32),
            pltpu.VMEM((2, BK, 16), jnp.float32),
            pltpu.VMEM((2, BK, 16), jnp.float32),
            pltpu.VMEM((BK, 16), jnp.float32),
            pltpu.VMEM((WBC, 16), jnp.float32),
            pltpu.VMEM_SHARED((M, 16), jnp.float32),
            pltpu.SemaphoreType.DMA,
            pltpu.SemaphoreType.DMA,
            pltpu.SemaphoreType.DMA,
            pltpu.SemaphoreType.DMA,
        ],
        compiler_params=pltpu.CompilerParams(use_tc_tiling_on_sc=False),
    )
    return f(hpq.reshape(NS * M, 2 * SW), edq.reshape(NS * M, SW),
             es16, ed16, src, dst)


def kernel(inputs, edge_index, W_in, b_in, spatial_emb, temporal_emb, Wg, a_src,
           a_dst, W_agg, b_agg, W_out1, b_out1, W_out2, b_out2):
    src, dst = edge_index[0], edge_index[1]
    x = _embed(inputs, W_in, b_in, spatial_emb, temporal_emb)

    last = None
    left = 0
    for i, right in enumerate(CKPNT):
        if i == 0:
            h0 = x[:, left:right]
        else:
            h0 = jnp.concatenate([last[:, None], x[:, left:right]], axis=1)
        res = h0.reshape(M, D)

        cur = res
        den2 = jnp.zeros((2, M, 16), jnp.float32)
        for l in range(L):
            hpq, edq, es16, ed16 = _proj_tables(
                cur, den2, Wg[l], a_src[l], a_dst[l], finish=(l > 0))
            out8, den2 = _edge_pass(hpq, edq, es16, ed16, src, dst)
            cur = out8.transpose(1, 0, 2).reshape(M, D)
        last = _combine(cur.reshape(B, S * N, D), den2, h0, W_agg, b_agg)
        left = right

    out = _decode(last, W_out1, b_out1, W_out2, b_out2)
    return out.reshape(B, SEQ_OUT, N, 1)


# speedup vs baseline: 26.6286x; 1.0517x over previous
import functools

import jax
import jax.numpy as jnp
from jax import lax
from jax.experimental import pallas as pl
from jax.experimental.pallas import tpu as pltpu
from jax.experimental.pallas import tpu_sc as plsc

B, T, N, DIN = 8, 13, 1200, 2
D, H, L, S = 128, 4, 2, 4
DH = D // H
SEQ_OUT = 12
M = B * S * N
E = 614400
CKPNT = [4, 7, 10, 13]

NS = 8
SW = 16
CHUNK = 1280
BK = 128
EPT_S = E // 16
EPT_D = E // 32
ROWS_PT = M // 16
WBC = 600

_sc_mesh = plsc.VectorSubcoreMesh(core_axis_name="c", subcore_axis_name="s")


def _embed_body(in_ref, w_ref, b_ref, sp_ref, te_ref, o_ref):
    t = pl.program_id(0) % T
    x2 = in_ref[0]
    w = w_ref[...]
    onehot = (lax.broadcasted_iota(jnp.int32, (T, 1), 0) == t).astype(jnp.float32)
    te_col = jnp.dot(te_ref[...], onehot,
                     preferred_element_type=jnp.float32)
    x = (x2[:, 0:1] * w[0:1, :] + x2[:, 1:2] * w[1:2, :]
         + b_ref[...][None, :] + sp_ref[...] + te_col)
    o_ref[0] = x


def _embed(inputs, W_in, b_in, spatial_emb, temporal_emb):
    flat_in = inputs.reshape(B * T, N, DIN)
    out = pl.pallas_call(
        _embed_body,
        grid=(B * T,),
        in_specs=[
            pl.BlockSpec((1, N, DIN), lambda i: (i, 0, 0)),
            pl.BlockSpec((DIN, D), lambda i: (0, 0)),
            pl.BlockSpec((D,), lambda i: (0,)),
            pl.BlockSpec((N, D), lambda i: (0, 0)),
            pl.BlockSpec((N, T), lambda i: (0, 0)),
        ],
        out_specs=pl.BlockSpec((1, N, D), lambda i: (i, 0, 0)),
        out_shape=jax.ShapeDtypeStruct((B * T, N, D), jnp.float32),
    )(flat_in, W_in, b_in, spatial_emb, temporal_emb)
    return out.reshape(B, T, N, D)


def _den_expand(den16):
    hsel = lax.broadcasted_iota(jnp.int32, (16, D), 0)
    csel = lax.broadcasted_iota(jnp.int32, (16, D), 1)
    q = (hsel == csel // DH).astype(jnp.float32)
    return jnp.dot(den16, q, preferred_element_type=jnp.float32)


_PROJ_R = 768


def _proj_body(finish, h_ref, den_ref, wg_ref, av_ref,
               hpq_ref, edq_ref, es_ref, ed_ref):
    h = h_ref[...]
    if finish:
        h = h / (_den_expand(den_ref[0] + den_ref[1]) + 1e-16)
        h = jnp.where(h > 0, h, jnp.exp(jnp.minimum(h, 0.0)) - 1.0)
    hp = jnp.dot(h, wg_ref[...], preferred_element_type=jnp.float32)
    row = lax.broadcasted_iota(jnp.int32, (D, 16), 0)
    col = lax.broadcasted_iota(jnp.int32, (D, 16), 1)
    seg = ((row // DH == col) & (col < H)).astype(jnp.float32)
    av = av_ref[...]
    es = jnp.dot(hp * av[0:1, :], seg, preferred_element_type=jnp.float32)
    ed = jnp.dot(hp * av[1:2, :], seg, preferred_element_type=jnp.float32)
    es_ref[...] = es
    ed_ref[...] = ed
    r16 = lax.broadcasted_iota(jnp.int32, (16, 16), 0)
    for s in range(NS):
        sel = (r16 == s // 2).astype(jnp.float32)
        es_rep = jnp.dot(es, sel, preferred_element_type=jnp.float32)
        ed_rep = jnp.dot(ed, sel, preferred_element_type=jnp.float32)
        hpq_ref[s] = jnp.concatenate([hp[:, s * SW:(s + 1) * SW], es_rep], axis=1)
        edq_ref[s] = ed_rep


def _proj_tables(h, den2, Wg_l, a_src_l, a_dst_l, finish):
    av = jnp.stack([a_src_l.reshape(D), a_dst_l.reshape(D)])
    return pl.pallas_call(
        functools.partial(_proj_body, finish),
        grid=(M // _PROJ_R,),
        in_specs=[
            pl.BlockSpec((_PROJ_R, D), lambda i: (i, 0)),
            pl.BlockSpec((2, _PROJ_R, 16), lambda i: (0, i, 0)),
            pl.BlockSpec((D, D), lambda i: (0, 0)),
            pl.BlockSpec((2, D), lambda i: (0, 0)),
        ],
        out_specs=[
            pl.BlockSpec((NS, _PROJ_R, 2 * SW), lambda i: (0, i, 0)),
            pl.BlockSpec((NS, _PROJ_R, SW), lambda i: (0, i, 0)),
            pl.BlockSpec((_PROJ_R, 16), lambda i: (i, 0)),
            pl.BlockSpec((_PROJ_R, 16), lambda i: (i, 0)),
        ],
        out_shape=[
            jax.ShapeDtypeStruct((NS, M, 2 * SW), jnp.float32),
            jax.ShapeDtypeStruct((NS, M, SW), jnp.float32),
            jax.ShapeDtypeStruct((M, 16), jnp.float32),
            jax.ShapeDtypeStruct((M, 16), jnp.float32),
        ],
    )(h, den2, Wg_l, av)


def _combine_body(ret_ref, den_ref, res_ref, wagg_ref, bagg_ref, o_ref):
    ret = ret_ref[0] / (_den_expand(den_ref[0, 0] + den_ref[1, 0]) + 1e-16)
    x = ret.reshape(S, N, D) + res_ref[0]
    wv = wagg_ref[...]
    score = jnp.tanh(jnp.sum(x * wv[None, :, :], axis=-1) + bagg_ref[0])
    mx = jnp.max(score, axis=0, keepdims=True)
    ex = jnp.exp(score - mx)
    w = ex / jnp.sum(ex, axis=0, keepdims=True)
    agg = jnp.sum(x * w[:, :, None], axis=0)
    mu = jnp.mean(agg)
    var = jnp.mean((agg - mu) ** 2)
    o_ref[0] = (agg - mu) * lax.rsqrt(var + 1e-5)


def _combine(ret, den2, res, W_agg, b_agg):
    return pl.pallas_call(
        _combine_body,
        grid=(B,),
        in_specs=[
            pl.BlockSpec((1, S * N, D), lambda i: (i, 0, 0)),
            pl.BlockSpec((2, 1, S * N, 16), lambda i: (0, i, 0, 0)),
            pl.BlockSpec((1, S, N, D), lambda i: (i, 0, 0, 0)),
            pl.BlockSpec((1, D), lambda i: (0, 0)),
            pl.BlockSpec(memory_space=pltpu.SMEM),
        ],
        out_specs=pl.BlockSpec((1, N, D), lambda i: (i, 0, 0)),
        out_shape=jax.ShapeDtypeStruct((B, N, D), jnp.float32),
    )(ret.reshape(B, S * N, D), den2.reshape(2, B, S * N, 16), res,
      W_agg.reshape(1, D), b_agg)


def _decode_body(last_ref, w1_ref, b1_ref, w2_ref, b2_ref, o_ref):
    x = last_ref[0]
    w1 = w1_ref[...]
    b1 = b1_ref[...]
    w2 = w2_ref[...]
    for j in range(SEQ_OUT):
        rj = jnp.maximum(x * w1[0, j] + b1[0, j], 0.0)
        o_ref[0, j, :] = jnp.sum(rj * w2, axis=-1) + b2_ref[0]


def _decode(last, W_out1, b_out1, W_out2, b_out2):
    return pl.pallas_call(
        _decode_body,
        grid=(B,),
        in_specs=[
            pl.BlockSpec((1, N, D), lambda i: (i, 0, 0)),
            pl.BlockSpec((1, SEQ_OUT), lambda i: (0, 0)),
            pl.BlockSpec((1, SEQ_OUT), lambda i: (0, 0)),
            pl.BlockSpec((1, D), lambda i: (0, 0)),
            pl.BlockSpec(memory_space=pltpu.SMEM),
        ],
        out_specs=pl.BlockSpec((1, SEQ_OUT, N), lambda i: (i, 0, 0)),
        out_shape=jax.ShapeDtypeStruct((B, SEQ_OUT, N), jnp.float32),
    )(last, W_out1, b_out1.reshape(1, SEQ_OUT), W_out2.reshape(1, D), b_out2)


def _edge_body(hpq_hbm, edq_hbm, es_hbm, ed_hbm, src_hbm, dst_hbm,
               out_hbm, den_hbm,
               srcc_v, idxe_v, dl_v, a_v, e_v, es2_v, sc_v, wb_v, acc_sh,
               sga0, sga1, sga2, sga3, sge0, sge1, sge2, sge3, ssc0, ssc1):
    cid = lax.axis_index("c")
    sid = lax.axis_index("s")

    def zero_wb():
        z = jnp.zeros((16,), jnp.float32)
        def zr(r, _):
            wb_v[r, pl.ds(0, 16)] = z
            return 0
        lax.fori_loop(0, WBC, zr, 0)

    def zero_acc():
        for i in range(ROWS_PT // WBC):
            pltpu.sync_copy(wb_v,
                            acc_sh.at[pl.ds(sid * ROWS_PT + i * WBC, WBC)])

    def stage_chunk(e0, tab_off, den_mode):
        pltpu.sync_copy(src_hbm.at[pl.ds(e0, CHUNK)], srcc_v)
        pltpu.sync_copy(dst_hbm.at[pl.ds(e0, CHUNK)], idxe_v)

        def fix(j, _):
            sv = srcc_v[pl.ds(j * 16, 16)]
            dv = idxe_v[pl.ds(j * 16, 16)]
            dl_v[j // 8, pl.ds((j % 8) * 16, 16)] = dv
            if not den_mode:
                srcc_v[pl.ds(j * 16, 16)] = sv + tab_off
                idxe_v[pl.ds(j * 16, 16)] = dv + tab_off
            return 0
        lax.fori_loop(0, CHUNK // 16, fix, 0)

    def run_pass(tab_off, n_chunks, base, den_mode):
        sga = [sga0, sga1, sga2, sga3]
        sge = [sge0, sge1, sge2, sge3]
        ssc = [ssc0, ssc1]
        nb = CHUNK // BK
        depth = 4

        def chunk(ci, _):
            stage_chunk(base + ci * CHUNK, tab_off, den_mode)
            hA = [None] * depth
            hE = [None] * depth
            hS = [None, None]

            def start(k):
                buf = k % depth
                sl = pl.ds(k * BK, BK)
                if den_mode:
                    hA[buf] = pltpu.async_copy(es_hbm.at[srcc_v.at[sl]],
                                               es2_v.at[buf], sga[buf])
                    hE[buf] = pltpu.async_copy(ed_hbm.at[idxe_v.at[sl]],
                                               e_v.at[buf], sge[buf])
                else:
                    hA[buf] = pltpu.async_copy(hpq_hbm.at[srcc_v.at[sl]],
                                               a_v.at[buf], sga[buf])
                    hE[buf] = pltpu.async_copy(edq_hbm.at[idxe_v.at[sl]],
                                               e_v.at[buf], sge[buf])

            for k in range(depth - 1):
                start(k)
            for k in range(nb):
                buf = k % depth
                sbuf = k % 2
                if k + depth - 1 < nb:
                    start(k + depth - 1)
                hA[buf].wait()
                hE[buf].wait()
                if hS[sbuf] is not None:
                    hS[sbuf].wait()

                def edge(e, _, buf=buf, sbuf=sbuf, den_mode=den_mode):
                    for u in range(2):
                        ee = e * 2 + u
                        if den_mode:
                            t = (es2_v[buf, ee, pl.ds(0, 16)]
                                 + e_v[buf, ee, pl.ds(0, 16)])
                            ex = jnp.exp(jnp.where(t >= 0.0, t, 0.2 * t))
                            sc_v[sbuf, ee, pl.ds(0, 16)] = ex
                        else:
                            t = (a_v[buf, ee, pl.ds(SW, 16)]
                                 + e_v[buf, ee, pl.ds(0, 16)])
                            ex = jnp.exp(jnp.where(t >= 0.0, t, 0.2 * t))
                            sc_v[sbuf, ee, pl.ds(0, 16)] = (
                                a_v[buf, ee, pl.ds(0, 16)] * ex)
                    return 0
                lax.fori_loop(0, BK // 2, edge, 0)

                hS[sbuf] = pltpu.async_copy(sc_v.at[sbuf],
                                            acc_sh.at[dl_v.at[k]],
                                            ssc[sbuf], add=True)
            for sbuf in range(2):
                if hS[sbuf] is not None:
                    hS[sbuf].wait()
            return 0
        lax.fori_loop(0, n_chunks, chunk, 0)

    def writeback(dst_ref):
        r0 = sid * ROWS_PT
        def wchunk(ci, _):
            pltpu.sync_copy(acc_sh.at[pl.ds(r0 + ci * WBC, WBC)], wb_v)
            pltpu.sync_copy(wb_v, dst_ref.at[pl.ds(r0 + ci * WBC, WBC)])
            return 0
        lax.fori_loop(0, ROWS_PT // WBC, wchunk, 0)

    zero_wb()
    for q in range(NS // 2):
        zero_acc()
        plsc.subcore_barrier()
        run_pass((cid * (NS // 2) + q) * M, EPT_S // CHUNK, sid * EPT_S,
                 den_mode=False)
        plsc.subcore_barrier()
        writeback(out_hbm.at[cid * (NS // 2) + q])
        plsc.subcore_barrier()
        zero_wb()
    zero_acc()
    plsc.subcore_barrier()
    run_pass(0, EPT_D // CHUNK, cid * (E // 2) + sid * EPT_D, den_mode=True)
    plsc.subcore_barrier()
    writeback(den_hbm.at[cid])


def _edge_pass(hpq, edq, es16, ed16, src, dst):
    f = pl.kernel(
        _edge_body,
        out_type=[
            jax.ShapeDtypeStruct((NS, M, SW), jnp.float32),
            jax.ShapeDtypeStruct((2, M, 16), jnp.float32),
        ],
        mesh=_sc_mesh,
        scratch_types=[
            pltpu.VMEM((CHUNK,), jnp.int32),
            pltpu.VMEM((CHUNK,), jnp.int32),
            pltpu.VMEM((CHUNK // BK, BK), jnp.int32),
            pltpu.VMEM((4, BK, 2 * SW), jnp.float32),
            pltpu.VMEM((4, BK, 16), jnp.float32),
            pltpu.VMEM((4, BK, 16), jnp.float32),
            pltpu.VMEM((2, BK, 16), jnp.float32),
            pltpu.VMEM((WBC, 16), jnp.float32),
            pltpu.VMEM_SHARED((M, 16), jnp.float32),
        ] + [pltpu.SemaphoreType.DMA] * 10,
        compiler_params=pltpu.CompilerParams(use_tc_tiling_on_sc=False),
    )
    return f(hpq.reshape(NS * M, 2 * SW), edq.reshape(NS * M, SW),
             es16, ed16, src, dst)


def kernel(inputs, edge_index, W_in, b_in, spatial_emb, temporal_emb, Wg, a_src,
           a_dst, W_agg, b_agg, W_out1, b_out1, W_out2, b_out2):
    src, dst = edge_index[0], edge_index[1]
    x = _embed(inputs, W_in, b_in, spatial_emb, temporal_emb)

    last = None
    left = 0
    for i, right in enumerate(CKPNT):
        if i == 0:
            h0 = x[:, left:right]
        else:
            h0 = jnp.concatenate([last[:, None], x[:, left:right]], axis=1)
        res = h0.reshape(M, D)

        cur = res
        den2 = jnp.zeros((2, M, 16), jnp.float32)
        for l in range(L):
            hpq, edq, es16, ed16 = _proj_tables(
                cur, den2, Wg[l], a_src[l], a_dst[l], finish=(l > 0))
            out8, den2 = _edge_pass(hpq, edq, es16, ed16, src, dst)
            cur = out8.transpose(1, 0, 2).reshape(M, D)
        last = _combine(cur.reshape(B, S * N, D), den2, h0, W_agg, b_agg)
        left = right

    out = _decode(last, W_out1, b_out1, W_out2, b_out2)
    return out.reshape(B, SEQ_OUT, N, 1)


# speedup vs baseline: 51.8714x; 1.9480x over previous
import functools

import jax
import jax.numpy as jnp
from jax import lax
from jax.experimental import pallas as pl
from jax.experimental.pallas import tpu as pltpu
from jax.experimental.pallas import tpu_sc as plsc

B, T, N, DIN = 8, 13, 1200, 2
D, H, L, S = 128, 4, 2, 4
DH = D // H
SEQ_OUT = 12
M = B * S * N
E = 614400
CKPNT = [4, 7, 10, 13]

NS = 8
SW = 16
CHUNK = 1280
BK = 128
EPT_S = E // 16
EPT_D = E // 32
ROWS_PT = M // 16
WBC = 600

_sc_mesh = plsc.VectorSubcoreMesh(core_axis_name="c", subcore_axis_name="s")


def _embed_body(in_ref, w_ref, b_ref, sp_ref, te_ref, o_ref):
    t = pl.program_id(0) % T
    x2 = in_ref[0]
    w = w_ref[...]
    onehot = (lax.broadcasted_iota(jnp.int32, (T, 1), 0) == t).astype(jnp.float32)
    te_col = jnp.dot(te_ref[...], onehot,
                     preferred_element_type=jnp.float32)
    x = (x2[:, 0:1] * w[0:1, :] + x2[:, 1:2] * w[1:2, :]
         + b_ref[...][None, :] + sp_ref[...] + te_col)
    o_ref[0] = x


def _embed(inputs, W_in, b_in, spatial_emb, temporal_emb):
    flat_in = inputs.reshape(B * T, N, DIN)
    out = pl.pallas_call(
        _embed_body,
        grid=(B * T,),
        in_specs=[
            pl.BlockSpec((1, N, DIN), lambda i: (i, 0, 0)),
            pl.BlockSpec((DIN, D), lambda i: (0, 0)),
            pl.BlockSpec((D,), lambda i: (0,)),
            pl.BlockSpec((N, D), lambda i: (0, 0)),
            pl.BlockSpec((N, T), lambda i: (0, 0)),
        ],
        out_specs=pl.BlockSpec((1, N, D), lambda i: (i, 0, 0)),
        out_shape=jax.ShapeDtypeStruct((B * T, N, D), jnp.float32),
    )(flat_in, W_in, b_in, spatial_emb, temporal_emb)
    return out.reshape(B, T, N, D)


def _den_expand(den16):
    hsel = lax.broadcasted_iota(jnp.int32, (16, D), 0)
    csel = lax.broadcasted_iota(jnp.int32, (16, D), 1)
    q = (hsel == csel // DH).astype(jnp.float32)
    return jnp.dot(den16, q, preferred_element_type=jnp.float32)


_PROJ_R = 768


def _proj_body(finish, h_ref, den_ref, wg_ref, av_ref,
               hpq_ref, edq_ref, es_ref, ed_ref):
    h = h_ref[...]
    if finish:
        h = h / (_den_expand(den_ref[0] + den_ref[1]) + 1e-16)
        h = jnp.where(h > 0, h, jnp.exp(jnp.minimum(h, 0.0)) - 1.0)
    hp = jnp.dot(h, wg_ref[...], preferred_element_type=jnp.float32)
    row = lax.broadcasted_iota(jnp.int32, (D, 16), 0)
    col = lax.broadcasted_iota(jnp.int32, (D, 16), 1)
    seg = ((row // DH == col) & (col < H)).astype(jnp.float32)
    av = av_ref[...]
    es = jnp.dot(hp * av[0:1, :], seg, preferred_element_type=jnp.float32)
    ed = jnp.dot(hp * av[1:2, :], seg, preferred_element_type=jnp.float32)
    es_ref[...] = es
    ed_ref[...] = ed
    r16 = lax.broadcasted_iota(jnp.int32, (16, 16), 0)
    for s in range(NS):
        sel = (r16 == s // 2).astype(jnp.float32)
        es_rep = jnp.dot(es, sel, preferred_element_type=jnp.float32)
        ed_rep = jnp.dot(ed, sel, preferred_element_type=jnp.float32)
        hpq_ref[s] = jnp.concatenate([hp[:, s * SW:(s + 1) * SW], es_rep], axis=1)
        edq_ref[s] = ed_rep


def _proj_tables(h, den2, Wg_l, a_src_l, a_dst_l, finish):
    av = jnp.stack([a_src_l.reshape(D), a_dst_l.reshape(D)])
    return pl.pallas_call(
        functools.partial(_proj_body, finish),
        grid=(M // _PROJ_R,),
        in_specs=[
            pl.BlockSpec((_PROJ_R, D), lambda i: (i, 0)),
            pl.BlockSpec((2, _PROJ_R, 16), lambda i: (0, i, 0)),
            pl.BlockSpec((D, D), lambda i: (0, 0)),
            pl.BlockSpec((2, D), lambda i: (0, 0)),
        ],
        out_specs=[
            pl.BlockSpec((NS, _PROJ_R, 2 * SW), lambda i: (0, i, 0)),
            pl.BlockSpec((NS, _PROJ_R, SW), lambda i: (0, i, 0)),
            pl.BlockSpec((_PROJ_R, 16), lambda i: (i, 0)),
            pl.BlockSpec((_PROJ_R, 16), lambda i: (i, 0)),
        ],
        out_shape=[
            jax.ShapeDtypeStruct((NS, M, 2 * SW), jnp.float32),
            jax.ShapeDtypeStruct((NS, M, SW), jnp.float32),
            jax.ShapeDtypeStruct((M, 16), jnp.float32),
            jax.ShapeDtypeStruct((M, 16), jnp.float32),
        ],
    )(h, den2, Wg_l, av)


def _combine_body(ret_ref, den_ref, res_ref, wagg_ref, bagg_ref, o_ref):
    ret = ret_ref[0] / (_den_expand(den_ref[0, 0] + den_ref[1, 0]) + 1e-16)
    x = ret.reshape(S, N, D) + res_ref[0]
    wv = wagg_ref[...]
    score = jnp.tanh(jnp.sum(x * wv[None, :, :], axis=-1) + bagg_ref[0])
    mx = jnp.max(score, axis=0, keepdims=True)
    ex = jnp.exp(score - mx)
    w = ex / jnp.sum(ex, axis=0, keepdims=True)
    agg = jnp.sum(x * w[:, :, None], axis=0)
    mu = jnp.mean(agg)
    var = jnp.mean((agg - mu) ** 2)
    o_ref[0] = (agg - mu) * lax.rsqrt(var + 1e-5)


def _combine(ret, den2, res, W_agg, b_agg):
    return pl.pallas_call(
        _combine_body,
        grid=(B,),
        in_specs=[
            pl.BlockSpec((1, S * N, D), lambda i: (i, 0, 0)),
            pl.BlockSpec((2, 1, S * N, 16), lambda i: (0, i, 0, 0)),
            pl.BlockSpec((1, S, N, D), lambda i: (i, 0, 0, 0)),
            pl.BlockSpec((1, D), lambda i: (0, 0)),
            pl.BlockSpec(memory_space=pltpu.SMEM),
        ],
        out_specs=pl.BlockSpec((1, N, D), lambda i: (i, 0, 0)),
        out_shape=jax.ShapeDtypeStruct((B, N, D), jnp.float32),
    )(ret.reshape(B, S * N, D), den2.reshape(2, B, S * N, 16), res,
      W_agg.reshape(1, D), b_agg)


def _decode_body(last_ref, w1_ref, b1_ref, w2_ref, b2_ref, o_ref):
    x = last_ref[0]
    w1 = w1_ref[...]
    b1 = b1_ref[...]
    w2 = w2_ref[...]
    for j in range(SEQ_OUT):
        rj = jnp.maximum(x * w1[0, j] + b1[0, j], 0.0)
        o_ref[0, j, :] = jnp.sum(rj * w2, axis=-1) + b2_ref[0]


def _decode(last, W_out1, b_out1, W_out2, b_out2):
    return pl.pallas_call(
        _decode_body,
        grid=(B,),
        in_specs=[
            pl.BlockSpec((1, N, D), lambda i: (i, 0, 0)),
            pl.BlockSpec((1, SEQ_OUT), lambda i: (0, 0)),
            pl.BlockSpec((1, SEQ_OUT), lambda i: (0, 0)),
            pl.BlockSpec((1, D), lambda i: (0, 0)),
            pl.BlockSpec(memory_space=pltpu.SMEM),
        ],
        out_specs=pl.BlockSpec((1, SEQ_OUT, N), lambda i: (i, 0, 0)),
        out_shape=jax.ShapeDtypeStruct((B, SEQ_OUT, N), jnp.float32),
    )(last, W_out1, b_out1.reshape(1, SEQ_OUT), W_out2.reshape(1, D), b_out2)


def _den_body(es_hbm, ed_hbm, src_hbm, dst_hbm,
              den_hbm, exq_hbm,
              srcc_v, idxe_v, dl_v, e_v, es2_v, sc_v, wb_v, acc_sh,
              sga0, sga1, sga2, sga3, sge0, sge1, sge2, sge3,
              ssc0, ssc1, sxq0, sxq1):
    cid = lax.axis_index("c")
    sid = lax.axis_index("s")
    sga = [sga0, sga1, sga2, sga3]
    sge = [sge0, sge1, sge2, sge3]
    ssc = [ssc0, ssc1]
    sxq = [sxq0, sxq1]
    nb = CHUNK // BK
    depth = 4

    def zero_wb():
        z = jnp.zeros((16,), jnp.float32)
        def zr(r, _):
            wb_v[r, pl.ds(0, 16)] = z
            return 0
        lax.fori_loop(0, WBC, zr, 0)

    zero_wb()
    for i in range(ROWS_PT // WBC):
        pltpu.sync_copy(wb_v, acc_sh.at[pl.ds(sid * ROWS_PT + i * WBC, WBC)])
    plsc.subcore_barrier()

    base = cid * (E // 2) + sid * EPT_D

    def chunk(ci, _):
        e0 = base + ci * CHUNK
        pltpu.sync_copy(src_hbm.at[pl.ds(e0, CHUNK)], srcc_v)
        pltpu.sync_copy(dst_hbm.at[pl.ds(e0, CHUNK)], idxe_v)

        def fix(j, _):
            dl_v[j // 8, pl.ds((j % 8) * 16, 16)] = idxe_v[pl.ds(j * 16, 16)]
            return 0
        lax.fori_loop(0, CHUNK // 16, fix, 0)

        hA = [None] * depth
        hE = [None] * depth
        hS = [None, None]
        hX = [None, None]

        def start(k):
            buf = k % depth
            sl = pl.ds(k * BK, BK)
            hA[buf] = pltpu.async_copy(es_hbm.at[srcc_v.at[sl]],
                                       es2_v.at[buf], sga[buf])
            hE[buf] = pltpu.async_copy(ed_hbm.at[idxe_v.at[sl]],
                                       e_v.at[buf], sge[buf])

        for k in range(depth - 1):
            start(k)
        for k in range(nb):
            buf = k % depth
            sbuf = k % 2
            if k + depth - 1 < nb:
                start(k + depth - 1)
            hA[buf].wait()
            hE[buf].wait()
            if hS[sbuf] is not None:
                hS[sbuf].wait()
                hX[sbuf].wait()

            def edge(e, _, buf=buf, sbuf=sbuf):
                for u in range(2):
                    ee = e * 2 + u
                    t = es2_v[buf, ee, pl.ds(0, 16)] + e_v[buf, ee, pl.ds(0, 16)]
                    sc_v[sbuf, ee, pl.ds(0, 16)] = jnp.exp(
                        jnp.where(t >= 0.0, t, 0.2 * t))
                return 0
            lax.fori_loop(0, BK // 2, edge, 0)

            hS[sbuf] = pltpu.async_copy(sc_v.at[sbuf],
                                        acc_sh.at[dl_v.at[k]],
                                        ssc[sbuf], add=True)
            hX[sbuf] = pltpu.async_copy(sc_v.at[sbuf],
                                        exq_hbm.at[pl.ds(e0 + k * BK, BK)],
                                        sxq[sbuf])
        for sbuf in range(2):
            if hS[sbuf] is not None:
                hS[sbuf].wait()
                hX[sbuf].wait()
        return 0
    lax.fori_loop(0, EPT_D // CHUNK, chunk, 0)

    plsc.subcore_barrier()
    r0 = sid * ROWS_PT
    def wchunk(ci, _):
        pltpu.sync_copy(acc_sh.at[pl.ds(r0 + ci * WBC, WBC)], wb_v)
        pltpu.sync_copy(wb_v, den_hbm.at[cid, pl.ds(r0 + ci * WBC, WBC)])
        return 0
    lax.fori_loop(0, ROWS_PT // WBC, wchunk, 0)


def _slice_body(hpq_hbm, exq_hbm, src_hbm, dst_hbm,
                out_hbm,
                srcc_v, idxe_v, dl_v, a_v, x_v, sc_v, wb_v, acc_sh,
                sga0, sga1, sga2, sga3, sge0, sge1, sge2, sge3, ssc0, ssc1):
    cid = lax.axis_index("c")
    sid = lax.axis_index("s")
    sga = [sga0, sga1, sga2, sga3]
    sge = [sge0, sge1, sge2, sge3]
    ssc = [ssc0, ssc1]
    nb = CHUNK // BK
    depth = 4
    is_sc0 = cid == 0

    def zero_wb():
        z = jnp.zeros((16,), jnp.float32)
        def zr(r, _):
            wb_v[r, pl.ds(0, 16)] = z
            return 0
        lax.fori_loop(0, WBC, zr, 0)

    def zero_acc():
        for i in range(ROWS_PT // WBC):
            pltpu.sync_copy(wb_v, acc_sh.at[pl.ds(sid * ROWS_PT + i * WBC, WBC)])

    def run_pass(tab_off, head):
        base = sid * EPT_S

        def chunk(ci, _):
            e0 = base + ci * CHUNK
            pltpu.sync_copy(src_hbm.at[pl.ds(e0, CHUNK)], srcc_v)
            pltpu.sync_copy(dst_hbm.at[pl.ds(e0, CHUNK)], idxe_v)

            def fix(j, _):
                sv = srcc_v[pl.ds(j * 16, 16)]
                dl_v[j // 8, pl.ds((j % 8) * 16, 16)] = idxe_v[pl.ds(j * 16, 16)]
                srcc_v[pl.ds(j * 16, 16)] = sv + tab_off
                return 0
            lax.fori_loop(0, CHUNK // 16, fix, 0)

            hA = [None] * depth
            hE = [None] * depth
            hS = [None, None]

            def start(k):
                buf = k % depth
                hA[buf] = pltpu.async_copy(
                    hpq_hbm.at[srcc_v.at[pl.ds(k * BK, BK)]],
                    a_v.at[buf], sga[buf])
                hE[buf] = pltpu.async_copy(
                    exq_hbm.at[pl.ds(e0 + k * BK, BK)],
                    x_v.at[buf], sge[buf])

            for k in range(depth - 1):
                start(k)
            for k in range(nb):
                buf = k % depth
                sbuf = k % 2
                if k + depth - 1 < nb:
                    start(k + depth - 1)
                hA[buf].wait()
                hE[buf].wait()
                if hS[sbuf] is not None:
                    hS[sbuf].wait()

                def edge(e, _, buf=buf, sbuf=sbuf, head=head):
                    for u in range(2):
                        ee = e * 2 + u
                        xr = x_v[buf, ee, pl.ds(0, 16)]
                        sp0 = jnp.broadcast_to(xr[head], (16,))
                        sp1 = jnp.broadcast_to(xr[2 + head], (16,))
                        sp = jnp.where(is_sc0, sp0, sp1)
                        sc_v[sbuf, ee, pl.ds(0, 16)] = (
                            a_v[buf, ee, pl.ds(0, 16)] * sp)
                    return 0
                lax.fori_loop(0, BK // 2, edge, 0)

                hS[sbuf] = pltpu.async_copy(sc_v.at[sbuf],
                                            acc_sh.at[dl_v.at[k]],
                                            ssc[sbuf], add=True)
            for sbuf in range(2):
                if hS[sbuf] is not None:
                    hS[sbuf].wait()
            return 0
        lax.fori_loop(0, EPT_S // CHUNK, chunk, 0)

    def writeback(dst_ref):
        r0 = sid * ROWS_PT
        def wchunk(ci, _):
            pltpu.sync_copy(acc_sh.at[pl.ds(r0 + ci * WBC, WBC)], wb_v)
            pltpu.sync_copy(wb_v, dst_ref.at[pl.ds(r0 + ci * WBC, WBC)])
            return 0
        lax.fori_loop(0, ROWS_PT // WBC, wchunk, 0)

    zero_wb()
    for q in range(NS // 2):
        zero_acc()
        plsc.subcore_barrier()
        run_pass((cid * (NS // 2) + q) * M, q // 2)
        plsc.subcore_barrier()
        writeback(out_hbm.at[cid * (NS // 2) + q])
        plsc.subcore_barrier()
        zero_wb()


def _edge_pass(hpq, edq, es16, ed16, src, dst):
    fden = pl.kernel(
        _den_body,
        out_type=[
            jax.ShapeDtypeStruct((2, M, 16), jnp.float32),
            jax.ShapeDtypeStruct((E, 16), jnp.float32),
        ],
        mesh=_sc_mesh,
        scratch_types=[
            pltpu.VMEM((CHUNK,), jnp.int32),
            pltpu.VMEM((CHUNK,), jnp.int32),
            pltpu.VMEM((CHUNK // BK, BK), jnp.int32),
            pltpu.VMEM((4, BK, 16), jnp.float32),
            pltpu.VMEM((4, BK, 16), jnp.float32),
            pltpu.VMEM((2, BK, 16), jnp.float32),
            pltpu.VMEM((WBC, 16), jnp.float32),
            pltpu.VMEM_SHARED((M, 16), jnp.float32),
        ] + [pltpu.SemaphoreType.DMA] * 12,
        compiler_params=pltpu.CompilerParams(use_tc_tiling_on_sc=False),
    )
    den2, exq = fden(es16, ed16, src, dst)

    fslc = pl.kernel(
        _slice_body,
        out_type=jax.ShapeDtypeStruct((NS, M, SW), jnp.float32),
        mesh=_sc_mesh,
        scratch_types=[
            pltpu.VMEM((CHUNK,), jnp.int32),
            pltpu.VMEM((CHUNK,), jnp.int32),
            pltpu.VMEM((CHUNK // BK, BK), jnp.int32),
            pltpu.VMEM((4, BK, 2 * SW), jnp.float32),
            pltpu.VMEM((4, BK, 16), jnp.float32),
            pltpu.VMEM((2, BK, 16), jnp.float32),
            pltpu.VMEM((WBC, 16), jnp.float32),
            pltpu.VMEM_SHARED((M, 16), jnp.float32),
        ] + [pltpu.SemaphoreType.DMA] * 10,
        compiler_params=pltpu.CompilerParams(use_tc_tiling_on_sc=False),
    )
    out8 = fslc(hpq.reshape(NS * M, 2 * SW), exq, src, dst)
    return out8, den2


def kernel(inputs, edge_index, W_in, b_in, spatial_emb, temporal_emb, Wg, a_src,
           a_dst, W_agg, b_agg, W_out1, b_out1, W_out2, b_out2):
    src, dst = edge_index[0], edge_index[1]
    x = _embed(inputs, W_in, b_in, spatial_emb, temporal_emb)

    last = None
    left = 0
    for i, right in enumerate(CKPNT):
        if i == 0:
            h0 = x[:, left:right]
        else:
            h0 = jnp.concatenate([last[:, None], x[:, left:right]], axis=1)
        res = h0.reshape(M, D)

        cur = res
        den2 = jnp.zeros((2, M, 16), jnp.float32)
        for l in range(L):
            hpq, edq, es16, ed16 = _proj_tables(
                cur, den2, Wg[l], a_src[l], a_dst[l], finish=(l > 0))
            out8, den2 = _edge_pass(hpq, edq, es16, ed16, src, dst)
            cur = out8.transpose(1, 0, 2).reshape(M, D)
        last = _combine(cur.reshape(B, S * N, D), den2, h0, W_agg, b_agg)
        left = right

    out = _decode(last, W_out1, b_out1, W_out2, b_out2)
    return out.reshape(B, SEQ_OUT, N, 1)


# speedup vs baseline: 75.8002x; 1.4613x over previous
import functools

import jax
import jax.numpy as jnp
from jax import lax
from jax.experimental import pallas as pl
from jax.experimental.pallas import tpu as pltpu
from jax.experimental.pallas import tpu_sc as plsc

B, T, N, DIN = 8, 13, 1200, 2
D, H, L, S = 128, 4, 2, 4
DH = D // H
SEQ_OUT = 12
M = B * S * N
E = 614400
CKPNT = [4, 7, 10, 13]

NS = 8
SW = 16
CHUNK = 1280
BK = 128
EPT_S = E // 16
EPT_D = E // 32
ROWS_PT = M // 16
WBC = 600

_sc_mesh = plsc.VectorSubcoreMesh(core_axis_name="c", subcore_axis_name="s")


def _embed_body(in_ref, w_ref, b_ref, sp_ref, te_ref, o_ref):
    t = pl.program_id(0) % T
    x2 = in_ref[0]
    w = w_ref[...]
    onehot = (lax.broadcasted_iota(jnp.int32, (T, 1), 0) == t).astype(jnp.float32)
    te_col = jnp.dot(te_ref[...], onehot,
                     preferred_element_type=jnp.float32)
    x = (x2[:, 0:1] * w[0:1, :] + x2[:, 1:2] * w[1:2, :]
         + b_ref[...][None, :] + sp_ref[...] + te_col)
    o_ref[0] = x


def _embed(inputs, W_in, b_in, spatial_emb, temporal_emb):
    flat_in = inputs.reshape(B * T, N, DIN)
    out = pl.pallas_call(
        _embed_body,
        grid=(B * T,),
        in_specs=[
            pl.BlockSpec((1, N, DIN), lambda i: (i, 0, 0)),
            pl.BlockSpec((DIN, D), lambda i: (0, 0)),
            pl.BlockSpec((D,), lambda i: (0,)),
            pl.BlockSpec((N, D), lambda i: (0, 0)),
            pl.BlockSpec((N, T), lambda i: (0, 0)),
        ],
        out_specs=pl.BlockSpec((1, N, D), lambda i: (i, 0, 0)),
        out_shape=jax.ShapeDtypeStruct((B * T, N, D), jnp.float32),
    )(flat_in, W_in, b_in, spatial_emb, temporal_emb)
    return out.reshape(B, T, N, D)


def _den_expand(den16):
    hsel = lax.broadcasted_iota(jnp.int32, (16, D), 0)
    csel = lax.broadcasted_iota(jnp.int32, (16, D), 1)
    q = (hsel == csel // DH).astype(jnp.float32)
    return jnp.dot(den16, q, preferred_element_type=jnp.float32)


_PROJ_R = 768


def _proj_body(finish, h_ref, den_ref, wg_ref, av_ref,
               hpq_ref, es_ref, ed_ref):
    h = h_ref[...]
    if finish:
        h = h / (_den_expand(den_ref[0] + den_ref[1]) + 1e-16)
        h = jnp.where(h > 0, h, jnp.exp(jnp.minimum(h, 0.0)) - 1.0)
    hp = jnp.dot(h, wg_ref[...], preferred_element_type=jnp.float32)
    row = lax.broadcasted_iota(jnp.int32, (D, 16), 0)
    col = lax.broadcasted_iota(jnp.int32, (D, 16), 1)
    seg = ((row // DH == col) & (col < H)).astype(jnp.float32)
    av = av_ref[...]
    es = jnp.dot(hp * av[0:1, :], seg, preferred_element_type=jnp.float32)
    ed = jnp.dot(hp * av[1:2, :], seg, preferred_element_type=jnp.float32)
    es_ref[...] = es
    ed_ref[...] = ed
    for s in range(NS):
        hpq_ref[s] = hp[:, s * SW:(s + 1) * SW]


def _proj_tables(h, den2, Wg_l, a_src_l, a_dst_l, finish):
    av = jnp.stack([a_src_l.reshape(D), a_dst_l.reshape(D)])
    return pl.pallas_call(
        functools.partial(_proj_body, finish),
        grid=(M // _PROJ_R,),
        in_specs=[
            pl.BlockSpec((_PROJ_R, D), lambda i: (i, 0)),
            pl.BlockSpec((2, _PROJ_R, 16), lambda i: (0, i, 0)),
            pl.BlockSpec((D, D), lambda i: (0, 0)),
            pl.BlockSpec((2, D), lambda i: (0, 0)),
        ],
        out_specs=[
            pl.BlockSpec((NS, _PROJ_R, SW), lambda i: (0, i, 0)),
            pl.BlockSpec((_PROJ_R, 16), lambda i: (i, 0)),
            pl.BlockSpec((_PROJ_R, 16), lambda i: (i, 0)),
        ],
        out_shape=[
            jax.ShapeDtypeStruct((NS, M, SW), jnp.float32),
            jax.ShapeDtypeStruct((M, 16), jnp.float32),
            jax.ShapeDtypeStruct((M, 16), jnp.float32),
        ],
    )(h, den2, Wg_l, av)


def _combine_body(ret_ref, den_ref, res_ref, wagg_ref, bagg_ref, o_ref):
    ret = ret_ref[0] / (_den_expand(den_ref[0, 0] + den_ref[1, 0]) + 1e-16)
    x = ret.reshape(S, N, D) + res_ref[0]
    wv = wagg_ref[...]
    score = jnp.tanh(jnp.sum(x * wv[None, :, :], axis=-1) + bagg_ref[0])
    mx = jnp.max(score, axis=0, keepdims=True)
    ex = jnp.exp(score - mx)
    w = ex / jnp.sum(ex, axis=0, keepdims=True)
    agg = jnp.sum(x * w[:, :, None], axis=0)
    mu = jnp.mean(agg)
    var = jnp.mean((agg - mu) ** 2)
    o_ref[0] = (agg - mu) * lax.rsqrt(var + 1e-5)


def _combine(ret, den2, res, W_agg, b_agg):
    return pl.pallas_call(
        _combine_body,
        grid=(B,),
        in_specs=[
            pl.BlockSpec((1, S * N, D), lambda i: (i, 0, 0)),
            pl.BlockSpec((2, 1, S * N, 16), lambda i: (0, i, 0, 0)),
            pl.BlockSpec((1, S, N, D), lambda i: (i, 0, 0, 0)),
            pl.BlockSpec((1, D), lambda i: (0, 0)),
            pl.BlockSpec(memory_space=pltpu.SMEM),
        ],
        out_specs=pl.BlockSpec((1, N, D), lambda i: (i, 0, 0)),
        out_shape=jax.ShapeDtypeStruct((B, N, D), jnp.float32),
    )(ret.reshape(B, S * N, D), den2.reshape(2, B, S * N, 16), res,
      W_agg.reshape(1, D), b_agg)


def _decode_body(last_ref, w1_ref, b1_ref, w2_ref, b2_ref, o_ref):
    x = last_ref[0]
    w1 = w1_ref[...]
    b1 = b1_ref[...]
    w2 = w2_ref[...]
    for j in range(SEQ_OUT):
        rj = jnp.maximum(x * w1[0, j] + b1[0, j], 0.0)
        o_ref[0, j, :] = jnp.sum(rj * w2, axis=-1) + b2_ref[0]


def _decode(last, W_out1, b_out1, W_out2, b_out2):
    return pl.pallas_call(
        _decode_body,
        grid=(B,),
        in_specs=[
            pl.BlockSpec((1, N, D), lambda i: (i, 0, 0)),
            pl.BlockSpec((1, SEQ_OUT), lambda i: (0, 0)),
            pl.BlockSpec((1, SEQ_OUT), lambda i: (0, 0)),
            pl.BlockSpec((1, D), lambda i: (0, 0)),
            pl.BlockSpec(memory_space=pltpu.SMEM),
        ],
        out_specs=pl.BlockSpec((1, SEQ_OUT, N), lambda i: (i, 0, 0)),
        out_shape=jax.ShapeDtypeStruct((B, SEQ_OUT, N), jnp.float32),
    )(last, W_out1, b_out1.reshape(1, SEQ_OUT), W_out2.reshape(1, D), b_out2)


def _den_body(es_hbm, ed_hbm, src_hbm, dst_hbm,
              den_hbm, exq_hbm,
              srcc_v, idxe_v, dl_v, e_v, es2_v, sc_v, wb_v, acc_sh,
              sga0, sga1, sga2, sga3, sge0, sge1, sge2, sge3,
              ssc0, ssc1, sxq0, sxq1):
    cid = lax.axis_index("c")
    sid = lax.axis_index("s")
    sga = [sga0, sga1, sga2, sga3]
    sge = [sge0, sge1, sge2, sge3]
    ssc = [ssc0, ssc1]
    sxq = [sxq0, sxq1]
    nb = CHUNK // BK
    depth = 4

    def zero_wb():
        z = jnp.zeros((16,), jnp.float32)
        def zr(r, _):
            wb_v[r, pl.ds(0, 16)] = z
            return 0
        lax.fori_loop(0, WBC, zr, 0)

    zero_wb()
    for i in range(ROWS_PT // WBC):
        pltpu.sync_copy(wb_v, acc_sh.at[pl.ds(sid * ROWS_PT + i * WBC, WBC)])
    plsc.subcore_barrier()

    base = cid * (E // 2) + sid * EPT_D

    def chunk(ci, _):
        e0 = base + ci * CHUNK
        pltpu.sync_copy(src_hbm.at[pl.ds(e0, CHUNK)], srcc_v)
        pltpu.sync_copy(dst_hbm.at[pl.ds(e0, CHUNK)], idxe_v)

        def fix(j, _):
            dl_v[j // 8, pl.ds((j % 8) * 16, 16)] = idxe_v[pl.ds(j * 16, 16)]
            return 0
        lax.fori_loop(0, CHUNK // 16, fix, 0)

        hA = [None] * depth
        hE = [None] * depth
        hS = [None, None]
        hX = [None, None]

        def start(k):
            buf = k % depth
            sl = pl.ds(k * BK, BK)
            hA[buf] = pltpu.async_copy(es_hbm.at[srcc_v.at[sl]],
                                       es2_v.at[buf], sga[buf])
            hE[buf] = pltpu.async_copy(ed_hbm.at[idxe_v.at[sl]],
                                       e_v.at[buf], sge[buf])

        for k in range(depth - 1):
            start(k)
        for k in range(nb):
            buf = k % depth
            sbuf = k % 2
            if k + depth - 1 < nb:
                start(k + depth - 1)
            hA[buf].wait()
            hE[buf].wait()
            if hS[sbuf] is not None:
                hS[sbuf].wait()
                hX[sbuf].wait()

            def edge(e, _, buf=buf, sbuf=sbuf):
                for u in range(2):
                    ee = e * 2 + u
                    t = es2_v[buf, ee, pl.ds(0, 16)] + e_v[buf, ee, pl.ds(0, 16)]
                    sc_v[sbuf, ee, pl.ds(0, 16)] = jnp.exp(
                        jnp.where(t >= 0.0, t, 0.2 * t))
                return 0
            lax.fori_loop(0, BK // 2, edge, 0)

            hS[sbuf] = pltpu.async_copy(sc_v.at[sbuf],
                                        acc_sh.at[dl_v.at[k]],
                                        ssc[sbuf], add=True)
            hX[sbuf] = pltpu.async_copy(sc_v.at[sbuf],
                                        exq_hbm.at[pl.ds(e0 + k * BK, BK)],
                                        sxq[sbuf])
        for sbuf in range(2):
            if hS[sbuf] is not None:
                hS[sbuf].wait()
                hX[sbuf].wait()
        return 0
    lax.fori_loop(0, EPT_D // CHUNK, chunk, 0)

    plsc.subcore_barrier()
    r0 = sid * ROWS_PT
    def wchunk(ci, _):
        pltpu.sync_copy(acc_sh.at[pl.ds(r0 + ci * WBC, WBC)], wb_v)
        pltpu.sync_copy(wb_v, den_hbm.at[cid, pl.ds(r0 + ci * WBC, WBC)])
        return 0
    lax.fori_loop(0, ROWS_PT // WBC, wchunk, 0)


def _slice_body(hpq_hbm, exq_hbm, src_hbm, dst_hbm,
                out_hbm,
                srcc_v, idxe_v, dl_v, a_v, x_v, sc_v, wb_v, acc_sh,
                sga0, sga1, sga2, sga3, sge0, sge1, sge2, sge3, ssc0, ssc1):
    cid = lax.axis_index("c")
    sid = lax.axis_index("s")
    sga = [sga0, sga1, sga2, sga3]
    sge = [sge0, sge1, sge2, sge3]
    ssc = [ssc0, ssc1]
    nb = CHUNK // BK
    depth = 4
    is_sc0 = cid == 0

    def zero_wb():
        z = jnp.zeros((16,), jnp.float32)
        def zr(r, _):
            wb_v[r, pl.ds(0, 16)] = z
            return 0
        lax.fori_loop(0, WBC, zr, 0)

    def zero_acc():
        for i in range(ROWS_PT // WBC):
            pltpu.sync_copy(wb_v, acc_sh.at[pl.ds(sid * ROWS_PT + i * WBC, WBC)])

    def run_pass(tab_off, head):
        base = sid * EPT_S

        def chunk(ci, _):
            e0 = base + ci * CHUNK
            pltpu.sync_copy(src_hbm.at[pl.ds(e0, CHUNK)], srcc_v)
            pltpu.sync_copy(dst_hbm.at[pl.ds(e0, CHUNK)], idxe_v)

            def fix(j, _):
                sv = srcc_v[pl.ds(j * 16, 16)]
                dl_v[j // 8, pl.ds((j % 8) * 16, 16)] = idxe_v[pl.ds(j * 16, 16)]
                srcc_v[pl.ds(j * 16, 16)] = sv + tab_off
                return 0
            lax.fori_loop(0, CHUNK // 16, fix, 0)

            hA = [None] * depth
            hE = [None] * depth
            hS = [None, None]

            def start(k):
                buf = k % depth
                hA[buf] = pltpu.async_copy(
                    hpq_hbm.at[srcc_v.at[pl.ds(k * BK, BK)]],
                    a_v.at[buf], sga[buf])
                hE[buf] = pltpu.async_copy(
                    exq_hbm.at[pl.ds(e0 + k * BK, BK)],
                    x_v.at[buf], sge[buf])

            for k in range(depth - 1):
                start(k)
            for k in range(nb):
                buf = k % depth
                sbuf = k % 2
                if k + depth - 1 < nb:
                    start(k + depth - 1)
                hA[buf].wait()
                hE[buf].wait()
                if hS[sbuf] is not None:
                    hS[sbuf].wait()

                def edge(e, _, buf=buf, sbuf=sbuf, head=head):
                    for u in range(2):
                        ee = e * 2 + u
                        xr = x_v[buf, ee, pl.ds(0, 16)]
                        sp0 = jnp.broadcast_to(xr[head], (16,))
                        sp1 = jnp.broadcast_to(xr[2 + head], (16,))
                        sp = jnp.where(is_sc0, sp0, sp1)
                        sc_v[sbuf, ee, pl.ds(0, 16)] = (
                            a_v[buf, ee, pl.ds(0, 16)] * sp)
                    return 0
                lax.fori_loop(0, BK // 2, edge, 0)

                hS[sbuf] = pltpu.async_copy(sc_v.at[sbuf],
                                            acc_sh.at[dl_v.at[k]],
                                            ssc[sbuf], add=True)
            for sbuf in range(2):
                if hS[sbuf] is not None:
                    hS[sbuf].wait()
            return 0
        lax.fori_loop(0, EPT_S // CHUNK, chunk, 0)

    def writeback(dst_ref):
        r0 = sid * ROWS_PT
        def wchunk(ci, _):
            pltpu.sync_copy(acc_sh.at[pl.ds(r0 + ci * WBC, WBC)], wb_v)
            pltpu.sync_copy(wb_v, dst_ref.at[pl.ds(r0 + ci * WBC, WBC)])
            return 0
        lax.fori_loop(0, ROWS_PT // WBC, wchunk, 0)

    zero_wb()
    for q in range(NS // 2):
        zero_acc()
        plsc.subcore_barrier()
        run_pass((cid * (NS // 2) + q) * M, q // 2)
        plsc.subcore_barrier()
        writeback(out_hbm.at[cid * (NS // 2) + q])
        plsc.subcore_barrier()
        zero_wb()


def _edge_pass(hpq, es16, ed16, src, dst):
    fden = pl.kernel(
        _den_body,
        out_type=[
            jax.ShapeDtypeStruct((2, M, 16), jnp.float32),
            jax.ShapeDtypeStruct((E, 16), jnp.float32),
        ],
        mesh=_sc_mesh,
        scratch_types=[
            pltpu.VMEM((CHUNK,), jnp.int32),
            pltpu.VMEM((CHUNK,), jnp.int32),
            pltpu.VMEM((CHUNK // BK, BK), jnp.int32),
            pltpu.VMEM((4, BK, 16), jnp.float32),
            pltpu.VMEM((4, BK, 16), jnp.float32),
            pltpu.VMEM((2, BK, 16), jnp.float32),
            pltpu.VMEM((WBC, 16), jnp.float32),
            pltpu.VMEM_SHARED((M, 16), jnp.float32),
        ] + [pltpu.SemaphoreType.DMA] * 12,
        compiler_params=pltpu.CompilerParams(use_tc_tiling_on_sc=False),
    )
    den2, exq = fden(es16, ed16, src, dst)

    fslc = pl.kernel(
        _slice_body,
        out_type=jax.ShapeDtypeStruct((NS, M, SW), jnp.float32),
        mesh=_sc_mesh,
        scratch_types=[
            pltpu.VMEM((CHUNK,), jnp.int32),
            pltpu.VMEM((CHUNK,), jnp.int32),
            pltpu.VMEM((CHUNK // BK, BK), jnp.int32),
            pltpu.VMEM((4, BK, SW), jnp.float32),
            pltpu.VMEM((4, BK, 16), jnp.float32),
            pltpu.VMEM((2, BK, 16), jnp.float32),
            pltpu.VMEM((WBC, 16), jnp.float32),
            pltpu.VMEM_SHARED((M, 16), jnp.float32),
        ] + [pltpu.SemaphoreType.DMA] * 10,
        compiler_params=pltpu.CompilerParams(use_tc_tiling_on_sc=False),
    )
    out8 = fslc(hpq.reshape(NS * M, SW), exq, src, dst)
    return out8, den2


def kernel(inputs, edge_index, W_in, b_in, spatial_emb, temporal_emb, Wg, a_src,
           a_dst, W_agg, b_agg, W_out1, b_out1, W_out2, b_out2):
    src, dst = edge_index[0], edge_index[1]
    x = _embed(inputs, W_in, b_in, spatial_emb, temporal_emb)

    last = None
    left = 0
    for i, right in enumerate(CKPNT):
        if i == 0:
            h0 = x[:, left:right]
        else:
            h0 = jnp.concatenate([last[:, None], x[:, left:right]], axis=1)
        res = h0.reshape(M, D)

        cur = res
        den2 = jnp.zeros((2, M, 16), jnp.float32)
        for l in range(L):
            hpq, es16, ed16 = _proj_tables(
                cur, den2, Wg[l], a_src[l], a_dst[l], finish=(l > 0))
            out8, den2 = _edge_pass(hpq, es16, ed16, src, dst)
            cur = out8.transpose(1, 0, 2).reshape(M, D)
        last = _combine(cur.reshape(B, S * N, D), den2, h0, W_agg, b_agg)
        left = right

    out = _decode(last, W_out1, b_out1, W_out2, b_out2)
    return out.reshape(B, SEQ_OUT, N, 1)


# speedup vs baseline: 75.8187x; 1.0002x over previous
import functools

import jax
import jax.numpy as jnp
from jax import lax
from jax.experimental import pallas as pl
from jax.experimental.pallas import tpu as pltpu
from jax.experimental.pallas import tpu_sc as plsc

B, T, N, DIN = 8, 13, 1200, 2
D, H, L, S = 128, 4, 2, 4
DH = D // H
SEQ_OUT = 12
M = B * S * N
E = 614400
CKPNT = [4, 7, 10, 13]

NS = 8
SW = 16
CHUNK = 1280
BK = 128
EPT_S = E // 16
EPT_D = E // 32
ROWS_PT = M // 16
WBC = 600

_sc_mesh = plsc.VectorSubcoreMesh(core_axis_name="c", subcore_axis_name="s")


def _embed_body(in_ref, w_ref, b_ref, sp_ref, te_ref, o_ref):
    t = pl.program_id(0) % T
    x2 = in_ref[0]
    w = w_ref[...]
    onehot = (lax.broadcasted_iota(jnp.int32, (T, 1), 0) == t).astype(jnp.float32)
    te_col = jnp.dot(te_ref[...], onehot,
                     preferred_element_type=jnp.float32)
    x = (x2[:, 0:1] * w[0:1, :] + x2[:, 1:2] * w[1:2, :]
         + b_ref[...][None, :] + sp_ref[...] + te_col)
    o_ref[0] = x


def _embed(inputs, W_in, b_in, spatial_emb, temporal_emb):
    flat_in = inputs.reshape(B * T, N, DIN)
    out = pl.pallas_call(
        _embed_body,
        grid=(B * T,),
        in_specs=[
            pl.BlockSpec((1, N, DIN), lambda i: (i, 0, 0)),
            pl.BlockSpec((DIN, D), lambda i: (0, 0)),
            pl.BlockSpec((D,), lambda i: (0,)),
            pl.BlockSpec((N, D), lambda i: (0, 0)),
            pl.BlockSpec((N, T), lambda i: (0, 0)),
        ],
        out_specs=pl.BlockSpec((1, N, D), lambda i: (i, 0, 0)),
        out_shape=jax.ShapeDtypeStruct((B * T, N, D), jnp.float32),
    )(flat_in, W_in, b_in, spatial_emb, temporal_emb)
    return out.reshape(B, T, N, D)


def _den_expand(den16):
    hsel = lax.broadcasted_iota(jnp.int32, (16, D), 0)
    csel = lax.broadcasted_iota(jnp.int32, (16, D), 1)
    q = (hsel == csel // DH).astype(jnp.float32)
    return jnp.dot(den16, q, preferred_element_type=jnp.float32)


_PROJ_R = 768


def _proj_body(finish, h_ref, den_ref, wg_ref, av_ref,
               hpq_ref, es_ref, ed_ref):
    h = h_ref[...]
    if finish:
        h = h / (_den_expand(den_ref[0] + den_ref[1]) + 1e-16)
        h = jnp.where(h > 0, h, jnp.exp(jnp.minimum(h, 0.0)) - 1.0)
    hp = jnp.dot(h, wg_ref[...], preferred_element_type=jnp.float32)
    row = lax.broadcasted_iota(jnp.int32, (D, 16), 0)
    col = lax.broadcasted_iota(jnp.int32, (D, 16), 1)
    seg = ((row // DH == col) & (col < H)).astype(jnp.float32)
    av = av_ref[...]
    es = jnp.dot(hp * av[0:1, :], seg, preferred_element_type=jnp.float32)
    ed = jnp.dot(hp * av[1:2, :], seg, preferred_element_type=jnp.float32)
    es_ref[...] = es
    ed_ref[...] = ed
    for s in range(NS):
        hpq_ref[s] = hp[:, s * SW:(s + 1) * SW]


def _proj_tables(h, den2, Wg_l, a_src_l, a_dst_l, finish):
    av = jnp.stack([a_src_l.reshape(D), a_dst_l.reshape(D)])
    return pl.pallas_call(
        functools.partial(_proj_body, finish),
        grid=(M // _PROJ_R,),
        in_specs=[
            pl.BlockSpec((_PROJ_R, D), lambda i: (i, 0)),
            pl.BlockSpec((2, _PROJ_R, 16), lambda i: (0, i, 0)),
            pl.BlockSpec((D, D), lambda i: (0, 0)),
            pl.BlockSpec((2, D), lambda i: (0, 0)),
        ],
        out_specs=[
            pl.BlockSpec((NS, _PROJ_R, SW), lambda i: (0, i, 0)),
            pl.BlockSpec((_PROJ_R, 16), lambda i: (i, 0)),
            pl.BlockSpec((_PROJ_R, 16), lambda i: (i, 0)),
        ],
        out_shape=[
            jax.ShapeDtypeStruct((NS, M, SW), jnp.float32),
            jax.ShapeDtypeStruct((M, 16), jnp.float32),
            jax.ShapeDtypeStruct((M, 16), jnp.float32),
        ],
    )(h, den2, Wg_l, av)


def _combine_body(ret_ref, den_ref, res_ref, wagg_ref, bagg_ref, o_ref):
    ret = ret_ref[0] / (_den_expand(den_ref[0, 0] + den_ref[1, 0]) + 1e-16)
    x = ret.reshape(S, N, D) + res_ref[0]
    wv = wagg_ref[...]
    score = jnp.tanh(jnp.sum(x * wv[None, :, :], axis=-1) + bagg_ref[0])
    mx = jnp.max(score, axis=0, keepdims=True)
    ex = jnp.exp(score - mx)
    w = ex / jnp.sum(ex, axis=0, keepdims=True)
    agg = jnp.sum(x * w[:, :, None], axis=0)
    mu = jnp.mean(agg)
    var = jnp.mean((agg - mu) ** 2)
    o_ref[0] = (agg - mu) / jnp.sqrt(var + 1e-5)


def _combine(ret, den2, res, W_agg, b_agg):
    return pl.pallas_call(
        _combine_body,
        grid=(B,),
        in_specs=[
            pl.BlockSpec((1, S * N, D), lambda i: (i, 0, 0)),
            pl.BlockSpec((2, 1, S * N, 16), lambda i: (0, i, 0, 0)),
            pl.BlockSpec((1, S, N, D), lambda i: (i, 0, 0, 0)),
            pl.BlockSpec((1, D), lambda i: (0, 0)),
            pl.BlockSpec(memory_space=pltpu.SMEM),
        ],
        out_specs=pl.BlockSpec((1, N, D), lambda i: (i, 0, 0)),
        out_shape=jax.ShapeDtypeStruct((B, N, D), jnp.float32),
    )(ret.reshape(B, S * N, D), den2.reshape(2, B, S * N, 16), res,
      W_agg.reshape(1, D), b_agg)


def _decode_body(last_ref, w1_ref, b1_ref, w2_ref, b2_ref, o_ref):
    x = last_ref[0]
    w1 = w1_ref[...]
    b1 = b1_ref[...]
    w2 = w2_ref[...]
    for j in range(SEQ_OUT):
        rj = jnp.maximum(x * w1[0, j] + b1[0, j], 0.0)
        o_ref[0, j, :] = jnp.sum(rj * w2, axis=-1) + b2_ref[0]


def _decode(last, W_out1, b_out1, W_out2, b_out2):
    return pl.pallas_call(
        _decode_body,
        grid=(B,),
        in_specs=[
            pl.BlockSpec((1, N, D), lambda i: (i, 0, 0)),
            pl.BlockSpec((1, SEQ_OUT), lambda i: (0, 0)),
            pl.BlockSpec((1, SEQ_OUT), lambda i: (0, 0)),
            pl.BlockSpec((1, D), lambda i: (0, 0)),
            pl.BlockSpec(memory_space=pltpu.SMEM),
        ],
        out_specs=pl.BlockSpec((1, SEQ_OUT, N), lambda i: (i, 0, 0)),
        out_shape=jax.ShapeDtypeStruct((B, SEQ_OUT, N), jnp.float32),
    )(last, W_out1, b_out1.reshape(1, SEQ_OUT), W_out2.reshape(1, D), b_out2)


def _den_body(es_hbm, ed_hbm, src_hbm, dst_hbm,
              den_hbm, exq_hbm,
              srcc_v, idxe_v, dl_v, e_v, es2_v, sc_v, wb_v, acc_sh,
              sga0, sga1, sga2, sga3, sge0, sge1, sge2, sge3,
              ssc0, ssc1, sxq0, sxq1):
    cid = lax.axis_index("c")
    sid = lax.axis_index("s")
    sga = [sga0, sga1, sga2, sga3]
    sge = [sge0, sge1, sge2, sge3]
    ssc = [ssc0, ssc1]
    sxq = [sxq0, sxq1]
    nb = CHUNK // BK
    depth = 4

    def zero_wb():
        z = jnp.zeros((16,), jnp.float32)
        def zr(r, _):
            wb_v[r, pl.ds(0, 16)] = z
            return 0
        lax.fori_loop(0, WBC, zr, 0)

    zero_wb()
    for i in range(ROWS_PT // WBC):
        pltpu.sync_copy(wb_v, acc_sh.at[pl.ds(sid * ROWS_PT + i * WBC, WBC)])
    plsc.subcore_barrier()

    base = cid * (E // 2) + sid * EPT_D

    def chunk(ci, _):
        e0 = base + ci * CHUNK
        pltpu.sync_copy(src_hbm.at[pl.ds(e0, CHUNK)], srcc_v)
        pltpu.sync_copy(dst_hbm.at[pl.ds(e0, CHUNK)], idxe_v)

        def fix(j, _):
            dl_v[j // 8, pl.ds((j % 8) * 16, 16)] = idxe_v[pl.ds(j * 16, 16)]
            return 0
        lax.fori_loop(0, CHUNK // 16, fix, 0)

        hA = [None] * depth
        hE = [None] * depth
        hS = [None, None]
        hX = [None, None]

        def start(k):
            buf = k % depth
            sl = pl.ds(k * BK, BK)
            hA[buf] = pltpu.async_copy(es_hbm.at[srcc_v.at[sl]],
                                       es2_v.at[buf], sga[buf])
            hE[buf] = pltpu.async_copy(ed_hbm.at[idxe_v.at[sl]],
                                       e_v.at[buf], sge[buf])

        for k in range(depth - 1):
            start(k)
        for k in range(nb):
            buf = k % depth
            sbuf = k % 2
            if k + depth - 1 < nb:
                start(k + depth - 1)
            hA[buf].wait()
            hE[buf].wait()
            if hS[sbuf] is not None:
                hS[sbuf].wait()
                hX[sbuf].wait()

            def edge(e, _, buf=buf, sbuf=sbuf):
                for u in range(2):
                    ee = e * 2 + u
                    t = es2_v[buf, ee, pl.ds(0, 16)] + e_v[buf, ee, pl.ds(0, 16)]
                    sc_v[sbuf, ee, pl.ds(0, 16)] = jnp.exp(
                        jnp.where(t >= 0.0, t, 0.2 * t))
                return 0
            lax.fori_loop(0, BK // 2, edge, 0)

            hS[sbuf] = pltpu.async_copy(sc_v.at[sbuf],
                                        acc_sh.at[dl_v.at[k]],
                                        ssc[sbuf], add=True)
            hX[sbuf] = pltpu.async_copy(sc_v.at[sbuf],
                                        exq_hbm.at[pl.ds(e0 + k * BK, BK)],
                                        sxq[sbuf])
        for sbuf in range(2):
            if hS[sbuf] is not None:
                hS[sbuf].wait()
                hX[sbuf].wait()
        return 0
    lax.fori_loop(0, EPT_D // CHUNK, chunk, 0)

    plsc.subcore_barrier()
    r0 = sid * ROWS_PT
    def wchunk(ci, _):
        pltpu.sync_copy(acc_sh.at[pl.ds(r0 + ci * WBC, WBC)], wb_v)
        pltpu.sync_copy(wb_v, den_hbm.at[cid, pl.ds(r0 + ci * WBC, WBC)])
        return 0
    lax.fori_loop(0, ROWS_PT // WBC, wchunk, 0)


def _slice_body(hpq_hbm, exq_hbm, src_hbm, dst_hbm,
                out_hbm,
                srcc_v, idxe_v, dl_v, a_v, x_v, sc_v, wb_v, acc_sh,
                sga0, sga1, sga2, sga3, sge0, sge1, sge2, sge3, ssc0, ssc1):
    cid = lax.axis_index("c")
    sid = lax.axis_index("s")
    sga = [sga0, sga1, sga2, sga3]
    sge = [sge0, sge1, sge2, sge3]
    ssc = [ssc0, ssc1]
    nb = CHUNK // BK
    depth = 4
    is_sc0 = cid == 0

    def zero_wb():
        z = jnp.zeros((16,), jnp.float32)
        def zr(r, _):
            wb_v[r, pl.ds(0, 16)] = z
            return 0
        lax.fori_loop(0, WBC, zr, 0)

    def zero_acc():
        for i in range(ROWS_PT // WBC):
            pltpu.sync_copy(wb_v, acc_sh.at[pl.ds(sid * ROWS_PT + i * WBC, WBC)])

    def run_pass(tab_off, head):
        base = sid * EPT_S

        def chunk(ci, _):
            e0 = base + ci * CHUNK
            pltpu.sync_copy(src_hbm.at[pl.ds(e0, CHUNK)], srcc_v)
            pltpu.sync_copy(dst_hbm.at[pl.ds(e0, CHUNK)], idxe_v)

            def fix(j, _):
                sv = srcc_v[pl.ds(j * 16, 16)]
                dl_v[j // 8, pl.ds((j % 8) * 16, 16)] = idxe_v[pl.ds(j * 16, 16)]
                srcc_v[pl.ds(j * 16, 16)] = sv + tab_off
                return 0
            lax.fori_loop(0, CHUNK // 16, fix, 0)

            hA = [None] * depth
            hE = [None] * depth
            hS = [None, None]

            def start(k):
                buf = k % depth
                hA[buf] = pltpu.async_copy(
                    hpq_hbm.at[srcc_v.at[pl.ds(k * BK, BK)]],
                    a_v.at[buf], sga[buf])
                hE[buf] = pltpu.async_copy(
                    exq_hbm.at[pl.ds(e0 + k * BK, BK)],
                    x_v.at[buf], sge[buf])

            for k in range(depth - 1):
                start(k)
            for k in range(nb):
                buf = k % depth
                sbuf = k % 2
                if k + depth - 1 < nb:
                    start(k + depth - 1)
                hA[buf].wait()
                hE[buf].wait()
                if hS[sbuf] is not None:
                    hS[sbuf].wait()

                def edge(e, _, buf=buf, sbuf=sbuf, head=head):
                    for u in range(2):
                        ee = e * 2 + u
                        xr = x_v[buf, ee, pl.ds(0, 16)]
                        sp0 = jnp.broadcast_to(xr[head], (16,))
                        sp1 = jnp.broadcast_to(xr[2 + head], (16,))
                        sp = jnp.where(is_sc0, sp0, sp1)
                        sc_v[sbuf, ee, pl.ds(0, 16)] = (
                            a_v[buf, ee, pl.ds(0, 16)] * sp)
                    return 0
                lax.fori_loop(0, BK // 2, edge, 0)

                hS[sbuf] = pltpu.async_copy(sc_v.at[sbuf],
                                            acc_sh.at[dl_v.at[k]],
                                            ssc[sbuf], add=True)
            for sbuf in range(2):
                if hS[sbuf] is not None:
                    hS[sbuf].wait()
            return 0
        lax.fori_loop(0, EPT_S // CHUNK, chunk, 0)

    def writeback(dst_ref):
        r0 = sid * ROWS_PT
        def wchunk(ci, _):
            pltpu.sync_copy(acc_sh.at[pl.ds(r0 + ci * WBC, WBC)], wb_v)
            pltpu.sync_copy(wb_v, dst_ref.at[pl.ds(r0 + ci * WBC, WBC)])
            return 0
        lax.fori_loop(0, ROWS_PT // WBC, wchunk, 0)

    zero_wb()
    for q in range(NS // 2):
        zero_acc()
        plsc.subcore_barrier()
        run_pass((cid * (NS // 2) + q) * M, q // 2)
        plsc.subcore_barrier()
        writeback(out_hbm.at[cid * (NS // 2) + q])
        plsc.subcore_barrier()
        zero_wb()


def _edge_pass(hpq, es16, ed16, src, dst):
    fden = pl.kernel(
        _den_body,
        out_type=[
            jax.ShapeDtypeStruct((2, M, 16), jnp.float32),
            jax.ShapeDtypeStruct((E, 16), jnp.float32),
        ],
        mesh=_sc_mesh,
        scratch_types=[
            pltpu.VMEM((CHUNK,), jnp.int32),
            pltpu.VMEM((CHUNK,), jnp.int32),
            pltpu.VMEM((CHUNK // BK, BK), jnp.int32),
            pltpu.VMEM((4, BK, 16), jnp.float32),
            pltpu.VMEM((4, BK, 16), jnp.float32),
            pltpu.VMEM((2, BK, 16), jnp.float32),
            pltpu.VMEM((WBC, 16), jnp.float32),
            pltpu.VMEM_SHARED((M, 16), jnp.float32),
        ] + [pltpu.SemaphoreType.DMA] * 12,
        compiler_params=pltpu.CompilerParams(use_tc_tiling_on_sc=False),
    )
    den2, exq = fden(es16, ed16, src, dst)

    fslc = pl.kernel(
        _slice_body,
        out_type=jax.ShapeDtypeStruct((NS, M, SW), jnp.float32),
        mesh=_sc_mesh,
        scratch_types=[
            pltpu.VMEM((CHUNK,), jnp.int32),
            pltpu.VMEM((CHUNK,), jnp.int32),
            pltpu.VMEM((CHUNK // BK, BK), jnp.int32),
            pltpu.VMEM((4, BK, SW), jnp.float32),
            pltpu.VMEM((4, BK, 16), jnp.float32),
            pltpu.VMEM((2, BK, 16), jnp.float32),
            pltpu.VMEM((WBC, 16), jnp.float32),
            pltpu.VMEM_SHARED((M, 16), jnp.float32),
        ] + [pltpu.SemaphoreType.DMA] * 10,
        compiler_params=pltpu.CompilerParams(use_tc_tiling_on_sc=False),
    )
    out8 = fslc(hpq.reshape(NS * M, SW), exq, src, dst)
    return out8, den2


def kernel(inputs, edge_index, W_in, b_in, spatial_emb, temporal_emb, Wg, a_src,
           a_dst, W_agg, b_agg, W_out1, b_out1, W_out2, b_out2):
    src, dst = edge_index[0], edge_index[1]
    x = _embed(inputs, W_in, b_in, spatial_emb, temporal_emb)

    last = None
    left = 0
    for i, right in enumerate(CKPNT):
        if i == 0:
            h0 = x[:, left:right]
        else:
            h0 = jnp.concatenate([last[:, None], x[:, left:right]], axis=1)
        res = h0.reshape(M, D)

        cur = res
        den2 = jnp.zeros((2, M, 16), jnp.float32)
        for l in range(L):
            hpq, es16, ed16 = _proj_tables(
                cur, den2, Wg[l], a_src[l], a_dst[l], finish=(l > 0))
            out8, den2 = _edge_pass(hpq, es16, ed16, src, dst)
            cur = out8.transpose(1, 0, 2).reshape(M, D)
        last = _combine(cur.reshape(B, S * N, D), den2, h0, W_agg, b_agg)
        left = right

    out = _decode(last, W_out1, b_out1, W_out2, b_out2)
    return out.reshape(B, SEQ_OUT, N, 1)
